# Initial kernel scaffold; baseline (speedup 1.0000x reference)
import functools

import jax
import jax.numpy as jnp
from jax import lax
from jax.experimental import pallas as pl
from jax.experimental.pallas import tpu as pltpu
from jax.experimental.pallas import tpu_sc as plsc

N = 10000
NP = 10240
F_IN = 128
RW1 = 80
RW2 = 16
DW = 16
K = 128
JPC = 4
SUP = K * JPC
NTILES = 32


def _make_edge_pass(ch, nrow, rw):
    mesh = plsc.VectorSubcoreMesh(core_axis_name="c", subcore_axis_name="s")

    @functools.partial(
        pl.kernel,
        out_type=jax.ShapeDtypeStruct((2, nrow, rw), jnp.float32),
        mesh=mesh,
        scratch_types=[
            pltpu.VMEM_SHARED((nrow, rw), jnp.float32),
            pltpu.VMEM((JPC, K), jnp.int32),
            pltpu.VMEM((JPC, K), jnp.int32),
            pltpu.VMEM((SUP, rw), jnp.float32),
            pltpu.VMEM((SUP, DW), jnp.float32),
            pltpu.SemaphoreType.DMA,
            pltpu.SemaphoreType.DMA,
        ],
    )
    def edge_pass(src_hbm, dst_hbm, tbl_hbm, dtbl_hbm, zero_hbm, out_hbm,
                  acc, sidx, didx, rows, drows, gsem, isem):
        cid = lax.axis_index("c")
        sid = lax.axis_index("s")
        wid = cid * 16 + sid

        zrows = nrow // 16
        pltpu.sync_copy(zero_hbm.at[pl.ds(sid * zrows, zrows)],
                        acc.at[pl.ds(sid * zrows, zrows)])
        plsc.subcore_barrier()

        iota = lax.iota(jnp.int32, 16)
        if rw == RW1:
            pats = [iota // 8 + 2 * cc for cc in range(4)]
        else:
            splat8 = jnp.zeros((16,), jnp.int32) + 8
            m_lt8 = iota < 8
            m_eq8 = iota == 8
            zero_v = jnp.zeros((16,), jnp.float32)

        @pl.loop(0, ch)
        def _chunk(g):
            blk = (wid * ch + g) * JPC
            ci = pltpu.async_copy(src_hbm.at[pl.ds(blk, JPC)], sidx, isem)
            cj = pltpu.async_copy(dst_hbm.at[pl.ds(blk, JPC)], didx, isem)
            ci.wait()
            cj.wait()
            cps = []
            for j in range(JPC):
                cps.append(pltpu.async_copy(
                    tbl_hbm.at[sidx.at[j]], rows.at[pl.ds(j * K, K)], gsem))
                cps.append(pltpu.async_copy(
                    dtbl_hbm.at[didx.at[j]], drows.at[pl.ds(j * K, K)], gsem))
            for cp in cps:
                cp.wait()

            @pl.loop(0, SUP)
            def _edge(e):
                dv = drows[e, pl.ds(0, 16)]
                if rw == RW1:
                    av = rows[e, pl.ds(64, 16)]
                    s = av + dv
                    w = jnp.exp(jnp.maximum(s, 0.2 * s))
                    for cc in range(4):
                        xwc = rows[e, pl.ds(cc * 16, 16)]
                        wb = w.at[pats[cc]].get(mode="promise_in_bounds")
                        rows[e, pl.ds(cc * 16, 16)] = xwc * wb
                    rows[e, pl.ds(64, 16)] = w
                else:
                    row = rows[e, pl.ds(0, 16)]
                    sv = row.at[splat8].get(mode="promise_in_bounds") + dv
                    w = jnp.exp(jnp.maximum(sv, 0.2 * sv))
                    out = jnp.where(m_lt8, w * row, jnp.where(m_eq8, w, zero_v))
                    rows[e, pl.ds(0, 16)] = out

            for j in range(JPC):
                pltpu.sync_copy(rows.at[pl.ds(j * K, K)],
                                acc.at[didx.at[j]], add=True)

        plsc.subcore_barrier()

        @pl.when(sid == 0)
        def _writeout():
            pltpu.sync_copy(acc, out_hbm.at[cid])

    return edge_pass


def _tc1(xp, w1, a1s, a1d):
    blk = 256

    def body(x_ref, w_ref, s_ref, d_ref, t_ref, dd_ref):
        xw = jnp.dot(x_ref[...], w_ref[...], preferred_element_type=jnp.float32)
        asrc = jnp.dot(xw, s_ref[...], preferred_element_type=jnp.float32)
        adst = jnp.dot(xw, d_ref[...], preferred_element_type=jnp.float32)
        z8 = jnp.zeros((blk, 8), jnp.float32)
        t_ref[...] = jnp.concatenate([xw, asrc, z8], axis=1)
        dd_ref[...] = jnp.concatenate([adst, z8], axis=1)

    return pl.pallas_call(
        body,
        grid=(NP // blk,),
        in_specs=[
            pl.BlockSpec((blk, F_IN), lambda i: (i, 0)),
            pl.BlockSpec((F_IN, 64), lambda i: (0, 0)),
            pl.BlockSpec((64, 8), lambda i: (0, 0)),
            pl.BlockSpec((64, 8), lambda i: (0, 0)),
        ],
        out_specs=[
            pl.BlockSpec((blk, RW1), lambda i: (i, 0)),
            pl.BlockSpec((blk, DW), lambda i: (i, 0)),
        ],
        out_shape=[
            jax.ShapeDtypeStruct((NP, RW1), jnp.float32),
            jax.ShapeDtypeStruct((NP, DW), jnp.float32),
        ],
    )(xp, w1, a1s, a1d)


def _tc2(acc1, b1, w2, as2, ad2):
    blk = 256

    def body(a_ref, b_ref, w_ref, s_ref, d_ref, t_ref, dd_ref):
        m = a_ref[0] + a_ref[1]
        msg = m[:, 0:64]
        den = m[:, 64:72]
        dex = jnp.broadcast_to(den[:, :, None], (blk, 8, 8)).reshape(blk, 64)
        h = msg / (dex + 1e-16) + b_ref[...]
        h = jnp.where(h > 0, h, jnp.exp(h) - 1.0)
        xw2 = jnp.dot(h, w_ref[...], preferred_element_type=jnp.float32)
        asrc2 = jnp.sum(xw2 * s_ref[...], axis=1, keepdims=True)
        adst2 = jnp.sum(xw2 * d_ref[...], axis=1, keepdims=True)
        z7 = jnp.zeros((blk, 7), jnp.float32)
        t_ref[...] = jnp.concatenate([xw2, asrc2, z7], axis=1)
        dd_ref[...] = jnp.broadcast_to(adst2, (blk, DW))

    return pl.pallas_call(
        body,
        grid=(NP // blk,),
        in_specs=[
            pl.BlockSpec((2, blk, RW1), lambda i: (0, i, 0)),
            pl.BlockSpec((1, 64), lambda i: (0, 0)),
            pl.BlockSpec((64, 8), lambda i: (0, 0)),
            pl.BlockSpec((1, 8), lambda i: (0, 0)),
            pl.BlockSpec((1, 8), lambda i: (0, 0)),
        ],
        out_specs=[
            pl.BlockSpec((blk, RW2), lambda i: (i, 0)),
            pl.BlockSpec((blk, DW), lambda i: (i, 0)),
        ],
        out_shape=[
            jax.ShapeDtypeStruct((NP, RW2), jnp.float32),
            jax.ShapeDtypeStruct((NP, DW), jnp.float32),
        ],
    )(acc1, b1, w2, as2, ad2)


def _tc3(acc2, b2):
    blk = 256

    def body(a_ref, b_ref, o_ref):
        m = a_ref[0] + a_ref[1]
        v = m[:, 0:8] / (m[:, 8:9] + 1e-16) + b_ref[...]
        mx = jnp.max(v, axis=1, keepdims=True)
        lse = mx + jnp.log(jnp.sum(jnp.exp(v - mx), axis=1, keepdims=True))
        o_ref[...] = v - lse

    return pl.pallas_call(
        body,
        grid=(NP // blk,),
        in_specs=[
            pl.BlockSpec((2, blk, RW2), lambda i: (0, i, 0)),
            pl.BlockSpec((1, 8), lambda i: (0, 0)),
        ],
        out_specs=pl.BlockSpec((blk, 8), lambda i: (i, 0)),
        out_shape=jax.ShapeDtypeStruct((NP, 8), jnp.float32),
    )(acc2, b2)


def kernel(x, edge_index, W1, att_src1, att_dst1, b1,
           W2, att_src2, att_dst2, b2):
    e = edge_index.shape[1]
    tot = e + N
    ch = -(-tot // (NTILES * SUP))
    ep = NTILES * SUP * ch

    loops = jnp.arange(N, dtype=jnp.int32)
    pad = jnp.full((ep - tot,), N, jnp.int32)
    src = jnp.concatenate([edge_index[0], loops, pad]).reshape(ep // K, K)
    dst = jnp.concatenate([edge_index[1], loops, pad]).reshape(ep // K, K)

    xp = jnp.pad(x, ((0, NP - N), (0, 0)))
    eye8 = jnp.eye(8, dtype=jnp.float32)
    a1s = (att_src1[:, :, None] * eye8[:, None, :]).reshape(64, 8)
    a1d = (att_dst1[:, :, None] * eye8[:, None, :]).reshape(64, 8)

    t1, d1 = _tc1(xp, W1, a1s, a1d)
    zero1 = jnp.zeros((NP, RW1), jnp.float32)
    acc1 = _make_edge_pass(ch, NP, RW1)(src, dst, t1, d1, zero1)

    t2, d2 = _tc2(acc1, b1.reshape(1, 64), W2,
                  att_src2.reshape(1, 8), att_dst2.reshape(1, 8))
    zero2 = jnp.zeros((NP, RW2), jnp.float32)
    acc2 = _make_edge_pass(ch, NP, RW2)(src, dst, t2, d2, zero2)

    out = _tc3(acc2, b2.reshape(1, 8))
    return out[:N]


# speedup vs baseline: 44.8911x; 44.8911x over previous
import dataclasses
import functools

import jax
import jax.numpy as jnp
from jax import lax
from jax.experimental import pallas as pl
from jax.experimental.pallas import tpu as pltpu
from jax.experimental.pallas import tpu_sc as plsc

N = 10000
NP = 10240
F_IN = 128
RW1 = 80
RW2 = 16
DW = 16
K = 128
JPC = 4
SUP = K * JPC
NTILES = 32


def _make_edge_pass(ch, nrow, rw):
    mesh = plsc.VectorSubcoreMesh(core_axis_name="c", subcore_axis_name="s")
    cp = pltpu.CompilerParams()
    if "needs_layout_passes" in pltpu.CompilerParams.__dataclass_fields__:
        cp = dataclasses.replace(cp, needs_layout_passes=False)
    if "use_tc_tiling_on_sc" in pltpu.CompilerParams.__dataclass_fields__:
        cp = dataclasses.replace(cp, use_tc_tiling_on_sc=False)

    @functools.partial(
        pl.kernel,
        out_type=jax.ShapeDtypeStruct((2, nrow, rw), jnp.float32),
        mesh=mesh,
        compiler_params=cp,
        scratch_types=[
            pltpu.VMEM_SHARED((nrow, rw), jnp.float32),
            pltpu.VMEM((JPC, K), jnp.int32),
            pltpu.VMEM((JPC, K), jnp.int32),
            pltpu.VMEM((SUP, rw), jnp.float32),
            pltpu.VMEM((SUP, DW), jnp.float32),
            pltpu.SemaphoreType.DMA,
            pltpu.SemaphoreType.DMA,
        ],
    )
    def edge_pass(src_hbm, dst_hbm, tbl_hbm, dtbl_hbm, zero_hbm, out_hbm,
                  acc, sidx, didx, rows, drows, gsem, isem):
        cid = lax.axis_index("c")
        sid = lax.axis_index("s")
        wid = cid * 16 + sid

        zrows = nrow // 16
        pltpu.sync_copy(zero_hbm.at[pl.ds(sid * zrows, zrows)],
                        acc.at[pl.ds(sid * zrows, zrows)])
        plsc.subcore_barrier()

        iota = lax.iota(jnp.int32, 16)
        if rw == RW1:
            pats = [iota // 8 + 2 * cc for cc in range(4)]
        else:
            splat8 = jnp.zeros((16,), jnp.int32) + 8
            m_lt8 = iota < 8
            m_eq8 = iota == 8
            zero_v = jnp.zeros((16,), jnp.float32)

        @pl.loop(0, ch)
        def _chunk(g):
            blk = (wid * ch + g) * JPC
            ci = pltpu.async_copy(src_hbm.at[pl.ds(blk, JPC)], sidx, isem)
            cj = pltpu.async_copy(dst_hbm.at[pl.ds(blk, JPC)], didx, isem)
            ci.wait()
            cj.wait()
            cps = []
            for j in range(JPC):
                cps.append(pltpu.async_copy(
                    tbl_hbm.at[sidx.at[j]], rows.at[pl.ds(j * K, K)], gsem))
                cps.append(pltpu.async_copy(
                    dtbl_hbm.at[didx.at[j]], drows.at[pl.ds(j * K, K)], gsem))
            for cp in cps:
                cp.wait()

            @pl.loop(0, SUP)
            def _edge(e):
                dv = drows[e, pl.ds(0, 16)]
                if rw == RW1:
                    av = rows[e, pl.ds(64, 16)]
                    s = av + dv
                    w = jnp.exp(jnp.maximum(s, 0.2 * s))
                    for cc in range(4):
                        xwc = rows[e, pl.ds(cc * 16, 16)]
                        wb = w.at[pats[cc]].get(mode="promise_in_bounds")
                        rows[e, pl.ds(cc * 16, 16)] = xwc * wb
                    rows[e, pl.ds(64, 16)] = w
                else:
                    row = rows[e, pl.ds(0, 16)]
                    sv = row.at[splat8].get(mode="promise_in_bounds") + dv
                    w = jnp.exp(jnp.maximum(sv, 0.2 * sv))
                    out = jnp.where(m_lt8, w * row, jnp.where(m_eq8, w, zero_v))
                    rows[e, pl.ds(0, 16)] = out

            for j in range(JPC):
                pltpu.sync_copy(rows.at[pl.ds(j * K, K)],
                                acc.at[didx.at[j]], add=True)

        plsc.subcore_barrier()

        @pl.when(sid == 0)
        def _writeout():
            pltpu.sync_copy(acc, out_hbm.at[cid])

    return edge_pass


def _tc1(xp, w1, a1s, a1d):
    blk = 256

    def body(x_ref, w_ref, s_ref, d_ref, t_ref, dd_ref):
        xw = jnp.dot(x_ref[...], w_ref[...], preferred_element_type=jnp.float32)
        asrc = jnp.dot(xw, s_ref[...], preferred_element_type=jnp.float32)
        adst = jnp.dot(xw, d_ref[...], preferred_element_type=jnp.float32)
        z8 = jnp.zeros((blk, 8), jnp.float32)
        t_ref[...] = jnp.concatenate([xw, asrc, z8], axis=1)
        dd_ref[...] = jnp.concatenate([adst, z8], axis=1)

    return pl.pallas_call(
        body,
        grid=(NP // blk,),
        in_specs=[
            pl.BlockSpec((blk, F_IN), lambda i: (i, 0)),
            pl.BlockSpec((F_IN, 64), lambda i: (0, 0)),
            pl.BlockSpec((64, 8), lambda i: (0, 0)),
            pl.BlockSpec((64, 8), lambda i: (0, 0)),
        ],
        out_specs=[
            pl.BlockSpec((blk, RW1), lambda i: (i, 0)),
            pl.BlockSpec((blk, DW), lambda i: (i, 0)),
        ],
        out_shape=[
            jax.ShapeDtypeStruct((NP, RW1), jnp.float32),
            jax.ShapeDtypeStruct((NP, DW), jnp.float32),
        ],
    )(xp, w1, a1s, a1d)


def _tc2(acc1, b1, w2, as2, ad2):
    blk = 256

    def body(a_ref, b_ref, w_ref, s_ref, d_ref, t_ref, dd_ref):
        m = a_ref[0] + a_ref[1]
        msg = m[:, 0:64]
        den = m[:, 64:72]
        dex = jnp.broadcast_to(den[:, :, None], (blk, 8, 8)).reshape(blk, 64)
        h = msg / (dex + 1e-16) + b_ref[...]
        h = jnp.where(h > 0, h, jnp.exp(h) - 1.0)
        xw2 = jnp.dot(h, w_ref[...], preferred_element_type=jnp.float32)
        asrc2 = jnp.sum(xw2 * s_ref[...], axis=1, keepdims=True)
        adst2 = jnp.sum(xw2 * d_ref[...], axis=1, keepdims=True)
        z7 = jnp.zeros((blk, 7), jnp.float32)
        t_ref[...] = jnp.concatenate([xw2, asrc2, z7], axis=1)
        dd_ref[...] = jnp.broadcast_to(adst2, (blk, DW))

    return pl.pallas_call(
        body,
        grid=(NP // blk,),
        in_specs=[
            pl.BlockSpec((2, blk, RW1), lambda i: (0, i, 0)),
            pl.BlockSpec((1, 64), lambda i: (0, 0)),
            pl.BlockSpec((64, 8), lambda i: (0, 0)),
            pl.BlockSpec((1, 8), lambda i: (0, 0)),
            pl.BlockSpec((1, 8), lambda i: (0, 0)),
        ],
        out_specs=[
            pl.BlockSpec((blk, RW2), lambda i: (i, 0)),
            pl.BlockSpec((blk, DW), lambda i: (i, 0)),
        ],
        out_shape=[
            jax.ShapeDtypeStruct((NP, RW2), jnp.float32),
            jax.ShapeDtypeStruct((NP, DW), jnp.float32),
        ],
    )(acc1, b1, w2, as2, ad2)


def _tc3(acc2, b2):
    blk = 256

    def body(a_ref, b_ref, o_ref):
        m = a_ref[0] + a_ref[1]
        v = m[:, 0:8] / (m[:, 8:9] + 1e-16) + b_ref[...]
        mx = jnp.max(v, axis=1, keepdims=True)
        lse = mx + jnp.log(jnp.sum(jnp.exp(v - mx), axis=1, keepdims=True))
        o_ref[...] = v - lse

    return pl.pallas_call(
        body,
        grid=(NP // blk,),
        in_specs=[
            pl.BlockSpec((2, blk, RW2), lambda i: (0, i, 0)),
            pl.BlockSpec((1, 8), lambda i: (0, 0)),
        ],
        out_specs=pl.BlockSpec((blk, 8), lambda i: (i, 0)),
        out_shape=jax.ShapeDtypeStruct((NP, 8), jnp.float32),
    )(acc2, b2)


def kernel(x, edge_index, W1, att_src1, att_dst1, b1,
           W2, att_src2, att_dst2, b2):
    e = edge_index.shape[1]
    tot = e + N
    ch = -(-tot // (NTILES * SUP))
    ep = NTILES * SUP * ch

    loops = jnp.arange(N, dtype=jnp.int32)
    pad = jnp.full((ep - tot,), N, jnp.int32)
    src = jnp.concatenate([edge_index[0], loops, pad]).reshape(ep // K, K)
    dst = jnp.concatenate([edge_index[1], loops, pad]).reshape(ep // K, K)

    xp = jnp.pad(x, ((0, NP - N), (0, 0)))
    eye8 = jnp.eye(8, dtype=jnp.float32)
    a1s = (att_src1[:, :, None] * eye8[:, None, :]).reshape(64, 8)
    a1d = (att_dst1[:, :, None] * eye8[:, None, :]).reshape(64, 8)

    t1, d1 = _tc1(xp, W1, a1s, a1d)
    zero1 = jnp.zeros((NP, RW1), jnp.float32)
    acc1 = _make_edge_pass(ch, NP, RW1)(src, dst, t1, d1, zero1)

    t2, d2 = _tc2(acc1, b1.reshape(1, 64), W2,
                  att_src2.reshape(1, 8), att_dst2.reshape(1, 8))
    zero2 = jnp.zeros((NP, RW2), jnp.float32)
    acc2 = _make_edge_pass(ch, NP, RW2)(src, dst, t2, d2, zero2)

    out = _tc3(acc2, b2.reshape(1, 8))
    return out[:N]


# speedup vs baseline: 55.0794x; 1.2270x over previous
import dataclasses
import functools

import jax
import jax.numpy as jnp
from jax import lax
from jax.experimental import pallas as pl
from jax.experimental.pallas import tpu as pltpu
from jax.experimental.pallas import tpu_sc as plsc

N = 10000
NP = 10240
F_IN = 128
RW1 = 80
RW2 = 16
DW = 16
K = 128
NTILES = 32


def _make_edge_pass(ch, nrow, rw, jpc):
    sup = K * jpc
    mesh = plsc.VectorSubcoreMesh(core_axis_name="c", subcore_axis_name="s")
    cp = pltpu.CompilerParams()
    if "needs_layout_passes" in pltpu.CompilerParams.__dataclass_fields__:
        cp = dataclasses.replace(cp, needs_layout_passes=False)
    if "use_tc_tiling_on_sc" in pltpu.CompilerParams.__dataclass_fields__:
        cp = dataclasses.replace(cp, use_tc_tiling_on_sc=False)

    nidx = ch * jpc

    @functools.partial(
        pl.kernel,
        out_type=jax.ShapeDtypeStruct((2, nrow, rw), jnp.float32),
        mesh=mesh,
        compiler_params=cp,
        scratch_types=[
            pltpu.VMEM_SHARED((nrow, rw), jnp.float32),
            pltpu.VMEM((nidx, K), jnp.int32),
            pltpu.VMEM((nidx, K), jnp.int32),
            pltpu.VMEM((sup, rw), jnp.float32),
            pltpu.VMEM((sup, rw), jnp.float32),
            pltpu.VMEM((sup, DW), jnp.float32),
            pltpu.VMEM((sup, DW), jnp.float32),
            pltpu.SemaphoreType.DMA,
            pltpu.SemaphoreType.DMA,
        ],
    )
    def edge_pass(src_hbm, dst_hbm, tbl_hbm, dtbl_hbm, zero_hbm, out_hbm,
                  acc, sidx, didx, rows_a, rows_b, drows_a, drows_b,
                  gsem, ssem):
        cid = lax.axis_index("c")
        sid = lax.axis_index("s")
        wid = cid * 16 + sid

        zrows = nrow // 16
        pltpu.sync_copy(zero_hbm.at[pl.ds(sid * zrows, zrows)],
                        acc.at[pl.ds(sid * zrows, zrows)])
        pltpu.sync_copy(src_hbm.at[pl.ds(wid * nidx, nidx)], sidx)
        pltpu.sync_copy(dst_hbm.at[pl.ds(wid * nidx, nidx)], didx)
        plsc.subcore_barrier()

        iota = lax.iota(jnp.int32, 16)
        if rw == RW1:
            pats = [iota // 8 + 2 * cc for cc in range(4)]
        else:
            splat8 = jnp.zeros((16,), jnp.int32) + 8
            m_lt8 = iota < 8
            m_eq8 = iota == 8
            zero_v = jnp.zeros((16,), jnp.float32)

        def g_fire(g, rows, drows):
            for j in range(jpc):
                r = g * jpc + j
                pltpu.async_copy(tbl_hbm.at[sidx.at[r]],
                                 rows.at[pl.ds(j * K, K)], gsem)
                pltpu.async_copy(dtbl_hbm.at[didx.at[r]],
                                 drows.at[pl.ds(j * K, K)], gsem)

        def g_drain(rows, drows):
            for j in range(jpc):
                pltpu.make_async_copy(tbl_hbm.at[pl.ds(0, K)],
                                      rows.at[pl.ds(j * K, K)], gsem).wait()
                pltpu.make_async_copy(dtbl_hbm.at[pl.ds(0, K)],
                                      drows.at[pl.ds(j * K, K)], gsem).wait()

        def s_start(g, rows):
            for j in range(jpc):
                r = g * jpc + j
                pltpu.async_copy(rows.at[pl.ds(j * K, K)],
                                 acc.at[didx.at[r]], ssem, add=True)

        def s_drain(rows):
            for j in range(jpc):
                pltpu.make_async_copy(rows.at[pl.ds(j * K, K)],
                                      acc.at[pl.ds(0, K)], ssem).wait()

        def compute(rows, drows):
            @plsc.parallel_loop(0, sup, unroll=2)
            def _edge(e):
                dv = drows[e, pl.ds(0, 16)]
                if rw == RW1:
                    av = rows[e, pl.ds(64, 16)]
                    s = av + dv
                    w = jnp.exp(jnp.maximum(s, 0.2 * s))
                    for cc in range(4):
                        xwc = rows[e, pl.ds(cc * 16, 16)]
                        wb = w.at[pats[cc]].get(mode="promise_in_bounds")
                        rows[e, pl.ds(cc * 16, 16)] = xwc * wb
                    rows[e, pl.ds(64, 16)] = w
                else:
                    row = rows[e, pl.ds(0, 16)]
                    sv = row.at[splat8].get(mode="promise_in_bounds") + dv
                    w = jnp.exp(jnp.maximum(sv, 0.2 * sv))
                    out = jnp.where(m_lt8, w * row, jnp.where(m_eq8, w, zero_v))
                    rows[e, pl.ds(0, 16)] = out

        g_fire(0, rows_a, drows_a)

        @pl.loop(0, ch // 2)
        def _pair(t):
            g0 = 2 * t
            g1 = g0 + 1

            @pl.when(t > 0)
            def _():
                s_drain(rows_b)

            g_fire(g1, rows_b, drows_b)
            g_drain(rows_a, drows_a)
            compute(rows_a, drows_a)
            s_start(g0, rows_a)
            g_drain(rows_b, drows_b)
            compute(rows_b, drows_b)
            s_drain(rows_a)

            @pl.when(g1 + 1 < ch)
            def _():
                g_fire(g1 + 1, rows_a, drows_a)

            s_start(g1, rows_b)

        s_drain(rows_b)
        plsc.subcore_barrier()

        @pl.when(sid == 0)
        def _writeout():
            pltpu.sync_copy(acc, out_hbm.at[cid])

    return edge_pass


def _tc1(xp, w1, a1s, a1d):
    blk = 256

    def body(x_ref, w_ref, s_ref, d_ref, t_ref, dd_ref):
        xw = jnp.dot(x_ref[...], w_ref[...], preferred_element_type=jnp.float32)
        asrc = jnp.dot(xw, s_ref[...], preferred_element_type=jnp.float32)
        adst = jnp.dot(xw, d_ref[...], preferred_element_type=jnp.float32)
        z8 = jnp.zeros((blk, 8), jnp.float32)
        t_ref[...] = jnp.concatenate([xw, asrc, z8], axis=1)
        dd_ref[...] = jnp.concatenate([adst, z8], axis=1)

    return pl.pallas_call(
        body,
        grid=(NP // blk,),
        in_specs=[
            pl.BlockSpec((blk, F_IN), lambda i: (i, 0)),
            pl.BlockSpec((F_IN, 64), lambda i: (0, 0)),
            pl.BlockSpec((64, 8), lambda i: (0, 0)),
            pl.BlockSpec((64, 8), lambda i: (0, 0)),
        ],
        out_specs=[
            pl.BlockSpec((blk, RW1), lambda i: (i, 0)),
            pl.BlockSpec((blk, DW), lambda i: (i, 0)),
        ],
        out_shape=[
            jax.ShapeDtypeStruct((NP, RW1), jnp.float32),
            jax.ShapeDtypeStruct((NP, DW), jnp.float32),
        ],
    )(xp, w1, a1s, a1d)


def _tc2(acc1, b1, w2, as2, ad2):
    blk = 256

    def body(a_ref, b_ref, w_ref, s_ref, d_ref, t_ref, dd_ref):
        m = a_ref[0] + a_ref[1]
        msg = m[:, 0:64]
        den = m[:, 64:72]
        dex = jnp.broadcast_to(den[:, :, None], (blk, 8, 8)).reshape(blk, 64)
        h = msg / (dex + 1e-16) + b_ref[...]
        h = jnp.where(h > 0, h, jnp.exp(h) - 1.0)
        xw2 = jnp.dot(h, w_ref[...], preferred_element_type=jnp.float32)
        asrc2 = jnp.sum(xw2 * s_ref[...], axis=1, keepdims=True)
        adst2 = jnp.sum(xw2 * d_ref[...], axis=1, keepdims=True)
        z7 = jnp.zeros((blk, 7), jnp.float32)
        t_ref[...] = jnp.concatenate([xw2, asrc2, z7], axis=1)
        dd_ref[...] = jnp.broadcast_to(adst2, (blk, DW))

    return pl.pallas_call(
        body,
        grid=(NP // blk,),
        in_specs=[
            pl.BlockSpec((2, blk, RW1), lambda i: (0, i, 0)),
            pl.BlockSpec((1, 64), lambda i: (0, 0)),
            pl.BlockSpec((64, 8), lambda i: (0, 0)),
            pl.BlockSpec((1, 8), lambda i: (0, 0)),
            pl.BlockSpec((1, 8), lambda i: (0, 0)),
        ],
        out_specs=[
            pl.BlockSpec((blk, RW2), lambda i: (i, 0)),
            pl.BlockSpec((blk, DW), lambda i: (i, 0)),
        ],
        out_shape=[
            jax.ShapeDtypeStruct((NP, RW2), jnp.float32),
            jax.ShapeDtypeStruct((NP, DW), jnp.float32),
        ],
    )(acc1, b1, w2, as2, ad2)


def _tc3(acc2, b2):
    blk = 256

    def body(a_ref, b_ref, o_ref):
        m = a_ref[0] + a_ref[1]
        v = m[:, 0:8] / (m[:, 8:9] + 1e-16) + b_ref[...]
        mx = jnp.max(v, axis=1, keepdims=True)
        lse = mx + jnp.log(jnp.sum(jnp.exp(v - mx), axis=1, keepdims=True))
        o_ref[...] = v - lse

    return pl.pallas_call(
        body,
        grid=(NP // blk,),
        in_specs=[
            pl.BlockSpec((2, blk, RW2), lambda i: (0, i, 0)),
            pl.BlockSpec((1, 8), lambda i: (0, 0)),
        ],
        out_specs=pl.BlockSpec((blk, 8), lambda i: (i, 0)),
        out_shape=jax.ShapeDtypeStruct((NP, 8), jnp.float32),
    )(acc2, b2)


def kernel(x, edge_index, W1, att_src1, att_dst1, b1,
           W2, att_src2, att_dst2, b2):
    e = edge_index.shape[1]
    tot = e + N
    jpc1, jpc2 = 2, 4
    ch1 = 2 * -(-tot // (2 * NTILES * K * jpc1))
    ch2 = 2 * -(-tot // (2 * NTILES * K * jpc2))
    ep = NTILES * K * max(jpc1 * ch1, jpc2 * ch2)

    loops = jnp.arange(N, dtype=jnp.int32)
    pad = jnp.full((ep - tot,), N, jnp.int32)
    src = jnp.concatenate([edge_index[0], loops, pad]).reshape(ep // K, K)
    dst = jnp.concatenate([edge_index[1], loops, pad]).reshape(ep // K, K)

    xp = jnp.pad(x, ((0, NP - N), (0, 0)))
    eye8 = jnp.eye(8, dtype=jnp.float32)
    a1s = (att_src1[:, :, None] * eye8[:, None, :]).reshape(64, 8)
    a1d = (att_dst1[:, :, None] * eye8[:, None, :]).reshape(64, 8)

    t1, d1 = _tc1(xp, W1, a1s, a1d)
    zero1 = jnp.zeros((NP, RW1), jnp.float32)
    acc1 = _make_edge_pass(ch1, NP, RW1, jpc1)(src, dst, t1, d1, zero1)

    t2, d2 = _tc2(acc1, b1.reshape(1, 64), W2,
                  att_src2.reshape(1, 8), att_dst2.reshape(1, 8))
    zero2 = jnp.zeros((NP, RW2), jnp.float32)
    acc2 = _make_edge_pass(ch2, NP, RW2, jpc2)(src, dst, t2, d2, zero2)

    out = _tc3(acc2, b2.reshape(1, 8))
    return out[:N]


# speedup vs baseline: 149.8951x; 2.7214x over previous
import dataclasses
import functools

import jax
import jax.numpy as jnp
from jax import lax
from jax.experimental import pallas as pl
from jax.experimental.pallas import tpu as pltpu
from jax.experimental.pallas import tpu_sc as plsc

N = 10000
NP = 10240
F_IN = 128
RW1 = 80
RW2 = 16
DW = 16
K = 128
NTILES = 32


def _make_edge_pass(ch, nrow, rw, jpc):
    sup = K * jpc
    mesh = plsc.VectorSubcoreMesh(core_axis_name="c", subcore_axis_name="s")
    cp = pltpu.CompilerParams()
    if "needs_layout_passes" in pltpu.CompilerParams.__dataclass_fields__:
        cp = dataclasses.replace(cp, needs_layout_passes=False)
    if "use_tc_tiling_on_sc" in pltpu.CompilerParams.__dataclass_fields__:
        cp = dataclasses.replace(cp, use_tc_tiling_on_sc=False)

    nidx = ch * jpc

    @functools.partial(
        pl.kernel,
        out_type=jax.ShapeDtypeStruct((2, nrow, rw), jnp.float32),
        mesh=mesh,
        compiler_params=cp,
        scratch_types=[
            pltpu.VMEM_SHARED((nrow, rw), jnp.float32),
            pltpu.VMEM((nidx, K), jnp.int32),
            pltpu.VMEM((nidx, K), jnp.int32),
            pltpu.VMEM((sup, rw), jnp.float32),
            pltpu.VMEM((sup, rw), jnp.float32),
            pltpu.VMEM((sup, DW), jnp.float32),
            pltpu.VMEM((sup, DW), jnp.float32),
            pltpu.SemaphoreType.DMA,
            pltpu.SemaphoreType.DMA,
        ],
    )
    def edge_pass(src_hbm, dst_hbm, tbl_hbm, dtbl_hbm, zero_hbm, out_hbm,
                  acc, sidx, didx, rows_a, rows_b, drows_a, drows_b,
                  gsem, ssem):
        cid = lax.axis_index("c")
        sid = lax.axis_index("s")
        wid = cid * 16 + sid

        zrows = nrow // 16
        pltpu.sync_copy(zero_hbm.at[pl.ds(sid * zrows, zrows)],
                        acc.at[pl.ds(sid * zrows, zrows)])
        pltpu.sync_copy(src_hbm.at[pl.ds(wid * nidx, nidx)], sidx)
        pltpu.sync_copy(dst_hbm.at[pl.ds(wid * nidx, nidx)], didx)
        plsc.subcore_barrier()

        iota = lax.iota(jnp.int32, 16)
        if rw == RW1:
            pats = [iota // 8 + 2 * cc for cc in range(4)]
        else:
            splat8 = jnp.zeros((16,), jnp.int32) + 8
            m_lt8 = iota < 8
            m_eq8 = iota == 8
            zero_v = jnp.zeros((16,), jnp.float32)

        def g_fire(g, rows, drows):
            for j in range(jpc):
                r = g * jpc + j
                pltpu.async_copy(tbl_hbm.at[sidx.at[r]],
                                 rows.at[pl.ds(j * K, K)], gsem)
                pltpu.async_copy(dtbl_hbm.at[didx.at[r]],
                                 drows.at[pl.ds(j * K, K)], gsem)

        def g_drain(rows, drows):
            for j in range(jpc):
                pltpu.make_async_copy(tbl_hbm.at[pl.ds(0, K)],
                                      rows.at[pl.ds(j * K, K)], gsem).wait()
                pltpu.make_async_copy(dtbl_hbm.at[pl.ds(0, K)],
                                      drows.at[pl.ds(j * K, K)], gsem).wait()

        def s_start(g, rows):
            for j in range(jpc):
                r = g * jpc + j
                pltpu.async_copy(rows.at[pl.ds(j * K, K)],
                                 acc.at[didx.at[r]], ssem, add=True)

        def s_drain(rows):
            for j in range(jpc):
                pltpu.make_async_copy(rows.at[pl.ds(j * K, K)],
                                      acc.at[pl.ds(0, K)], ssem).wait()

        def compute(rows, drows):
            @plsc.parallel_loop(0, sup, unroll=2)
            def _edge(e):
                dv = drows[e, pl.ds(0, 16)]
                if rw == RW1:
                    av = rows[e, pl.ds(64, 16)]
                    s = av + dv
                    w = jnp.exp(jnp.maximum(s, 0.2 * s))
                    for cc in range(4):
                        xwc = rows[e, pl.ds(cc * 16, 16)]
                        wb = w.at[pats[cc]].get(mode="promise_in_bounds")
                        rows[e, pl.ds(cc * 16, 16)] = xwc * wb
                    rows[e, pl.ds(64, 16)] = w
                else:
                    row = rows[e, pl.ds(0, 16)]
                    sv = row.at[splat8].get(mode="promise_in_bounds") + dv
                    w = jnp.exp(jnp.maximum(sv, 0.2 * sv))
                    out = jnp.where(m_lt8, w * row, jnp.where(m_eq8, w, zero_v))
                    rows[e, pl.ds(0, 16)] = out

        g_fire(0, rows_a, drows_a)

        @pl.loop(0, ch // 2)
        def _pair(t):
            g0 = 2 * t
            g1 = g0 + 1

            @pl.when(t > 0)
            def _():
                s_drain(rows_b)

            g_fire(g1, rows_b, drows_b)
            g_drain(rows_a, drows_a)
            compute(rows_a, drows_a)
            s_start(g0, rows_a)
            g_drain(rows_b, drows_b)
            compute(rows_b, drows_b)
            s_drain(rows_a)

            @pl.when(g1 + 1 < ch)
            def _():
                g_fire(g1 + 1, rows_a, drows_a)

            s_start(g1, rows_b)

        s_drain(rows_b)
        plsc.subcore_barrier()

        @pl.when(sid == 0)
        def _writeout():
            pltpu.sync_copy(acc, out_hbm.at[cid])

    return edge_pass


def _tc1(xp, w1, a1s, a1d):
    blk = 256

    def body(x_ref, w_ref, s_ref, d_ref, t_ref, dd_ref):
        xw = jnp.dot(x_ref[...], w_ref[...], preferred_element_type=jnp.float32)
        asrc = jnp.dot(xw, s_ref[...], preferred_element_type=jnp.float32)
        adst = jnp.dot(xw, d_ref[...], preferred_element_type=jnp.float32)
        z8 = jnp.zeros((blk, 8), jnp.float32)
        t_ref[...] = jnp.concatenate([xw, asrc, z8], axis=1)
        dd_ref[...] = jnp.concatenate([adst, z8], axis=1)

    return pl.pallas_call(
        body,
        grid=(NP // blk,),
        in_specs=[
            pl.BlockSpec((blk, F_IN), lambda i: (i, 0)),
            pl.BlockSpec((F_IN, 64), lambda i: (0, 0)),
            pl.BlockSpec((64, 8), lambda i: (0, 0)),
            pl.BlockSpec((64, 8), lambda i: (0, 0)),
        ],
        out_specs=[
            pl.BlockSpec((blk, RW1), lambda i: (i, 0)),
            pl.BlockSpec((blk, DW), lambda i: (i, 0)),
        ],
        out_shape=[
            jax.ShapeDtypeStruct((NP, RW1), jnp.float32),
            jax.ShapeDtypeStruct((NP, DW), jnp.float32),
        ],
    )(xp, w1, a1s, a1d)


def _tc2(acc1, b1, w2, as2, ad2):
    blk = 256

    def body(a_ref, b_ref, w_ref, s_ref, d_ref, t_ref, dd_ref):
        m = a_ref[0] + a_ref[1]
        msg = m[:, 0:64]
        den = m[:, 64:72]
        dex = jnp.broadcast_to(den[:, :, None], (blk, 8, 8)).reshape(blk, 64)
        h = msg / (dex + 1e-16) + b_ref[...]
        h = jnp.where(h > 0, h, jnp.exp(h) - 1.0)
        xw2 = jnp.dot(h, w_ref[...], preferred_element_type=jnp.float32)
        asrc2 = jnp.sum(xw2 * s_ref[...], axis=1, keepdims=True)
        adst2 = jnp.sum(xw2 * d_ref[...], axis=1, keepdims=True)
        z7 = jnp.zeros((blk, 7), jnp.float32)
        t_ref[...] = jnp.concatenate([xw2, asrc2, z7], axis=1)
        dd_ref[...] = jnp.broadcast_to(adst2, (blk, DW))

    return pl.pallas_call(
        body,
        grid=(NP // blk,),
        in_specs=[
            pl.BlockSpec((2, blk, RW1), lambda i: (0, i, 0)),
            pl.BlockSpec((1, 64), lambda i: (0, 0)),
            pl.BlockSpec((64, 8), lambda i: (0, 0)),
            pl.BlockSpec((1, 8), lambda i: (0, 0)),
            pl.BlockSpec((1, 8), lambda i: (0, 0)),
        ],
        out_specs=[
            pl.BlockSpec((blk, RW2), lambda i: (i, 0)),
            pl.BlockSpec((blk, DW), lambda i: (i, 0)),
        ],
        out_shape=[
            jax.ShapeDtypeStruct((NP, RW2), jnp.float32),
            jax.ShapeDtypeStruct((NP, DW), jnp.float32),
        ],
    )(acc1, b1, w2, as2, ad2)


def _tc3(acc2, b2):
    blk = 256

    def body(a_ref, b_ref, o_ref):
        m = a_ref[0] + a_ref[1]
        v = m[:, 0:8] / (m[:, 8:9] + 1e-16) + b_ref[...]
        mx = jnp.max(v, axis=1, keepdims=True)
        lse = mx + jnp.log(jnp.sum(jnp.exp(v - mx), axis=1, keepdims=True))
        o_ref[...] = v - lse

    return pl.pallas_call(
        body,
        grid=(NP // blk,),
        in_specs=[
            pl.BlockSpec((2, blk, RW2), lambda i: (0, i, 0)),
            pl.BlockSpec((1, 8), lambda i: (0, 0)),
        ],
        out_specs=pl.BlockSpec((blk, 8), lambda i: (i, 0)),
        out_shape=jax.ShapeDtypeStruct((NP, 8), jnp.float32),
    )(acc2, b2)


def kernel(x, edge_index, W1, att_src1, att_dst1, b1,
           W2, att_src2, att_dst2, b2):
    e = edge_index.shape[1]
    tot = e + N
    jpc1 = jpc2 = 2
    ch1 = ch2 = 2 * -(-tot // (2 * NTILES * K * jpc1))
    ep = NTILES * K * jpc1 * ch1

    loops = jnp.arange(N, dtype=jnp.int32)
    pad = N + jnp.arange(ep - tot, dtype=jnp.int32) % (NP - N)
    src = jnp.concatenate([edge_index[0], loops, pad]).reshape(ep // K, K)
    dst = jnp.concatenate([edge_index[1], loops, pad]).reshape(ep // K, K)

    xp = jnp.pad(x, ((0, NP - N), (0, 0)))
    eye8 = jnp.eye(8, dtype=jnp.float32)
    a1s = (att_src1[:, :, None] * eye8[:, None, :]).reshape(64, 8)
    a1d = (att_dst1[:, :, None] * eye8[:, None, :]).reshape(64, 8)

    t1, d1 = _tc1(xp, W1, a1s, a1d)
    zero1 = jnp.zeros((NP, RW1), jnp.float32)
    acc1 = _make_edge_pass(ch1, NP, RW1, jpc1)(src, dst, t1, d1, zero1)

    t2, d2 = _tc2(acc1, b1.reshape(1, 64), W2,
                  att_src2.reshape(1, 8), att_dst2.reshape(1, 8))
    zero2 = jnp.zeros((NP, RW2), jnp.float32)
    acc2 = _make_edge_pass(ch2, NP, RW2, jpc2)(src, dst, t2, d2, zero2)

    out = _tc3(acc2, b2.reshape(1, 8))
    return out[:N]


# speedup vs baseline: 173.4110x; 1.1569x over previous
import dataclasses
import functools

import jax
import jax.numpy as jnp
from jax import lax
from jax.experimental import pallas as pl
from jax.experimental.pallas import tpu as pltpu
from jax.experimental.pallas import tpu_sc as plsc

N = 10000
NP = 10240
F_IN = 128
RW1 = 80
RW2 = 16
DW = 16
K = 128
NTILES = 32


def _make_edge_pass(ch, nrow, rw, jpc):
    sup = K * jpc
    mesh = plsc.VectorSubcoreMesh(core_axis_name="c", subcore_axis_name="s")
    cp = pltpu.CompilerParams()
    if "needs_layout_passes" in pltpu.CompilerParams.__dataclass_fields__:
        cp = dataclasses.replace(cp, needs_layout_passes=False)
    if "use_tc_tiling_on_sc" in pltpu.CompilerParams.__dataclass_fields__:
        cp = dataclasses.replace(cp, use_tc_tiling_on_sc=False)

    nidx = ch * jpc

    @functools.partial(
        pl.kernel,
        out_type=[jax.ShapeDtypeStruct((nrow, rw), jnp.float32),
                  jax.ShapeDtypeStruct((nrow, rw), jnp.float32)],
        mesh=mesh,
        compiler_params=cp,
        scratch_types=[
            pltpu.VMEM_SHARED((nrow, rw), jnp.float32),
            pltpu.VMEM((nidx, K), jnp.int32),
            pltpu.VMEM((nidx, K), jnp.int32),
            pltpu.VMEM((sup, rw), jnp.float32),
            pltpu.VMEM((sup, rw), jnp.float32),
            pltpu.VMEM((sup, DW), jnp.float32),
            pltpu.VMEM((sup, DW), jnp.float32),
            pltpu.SemaphoreType.DMA,
            pltpu.SemaphoreType.DMA,
        ],
    )
    def edge_pass(src_hbm, dst_hbm, tbl_hbm, dtbl_hbm, zero_hbm,
                  out_a_hbm, out_b_hbm,
                  acc, sidx, didx, rows_a, rows_b, drows_a, drows_b,
                  gsem, ssem):
        cid = lax.axis_index("c")
        sid = lax.axis_index("s")
        wid = cid * 16 + sid

        zrows = nrow // 16
        pltpu.sync_copy(zero_hbm.at[pl.ds(sid * zrows, zrows)],
                        acc.at[pl.ds(sid * zrows, zrows)])
        pltpu.sync_copy(src_hbm.at[pl.ds(wid * nidx, nidx)], sidx)
        pltpu.sync_copy(dst_hbm.at[pl.ds(wid * nidx, nidx)], didx)
        plsc.subcore_barrier()

        iota = lax.iota(jnp.int32, 16)
        if rw == RW1:
            pats = [iota // 8 + 2 * cc for cc in range(4)]
        else:
            splat8 = jnp.zeros((16,), jnp.int32) + 8
            m_lt8 = iota < 8
            m_eq8 = iota == 8
            zero_v = jnp.zeros((16,), jnp.float32)

        def g_fire(g, rows, drows):
            for j in range(jpc):
                r = g * jpc + j
                pltpu.async_copy(tbl_hbm.at[sidx.at[r]],
                                 rows.at[pl.ds(j * K, K)], gsem)
                pltpu.async_copy(dtbl_hbm.at[didx.at[r]],
                                 drows.at[pl.ds(j * K, K)], gsem)

        def g_drain(rows, drows):
            for j in range(jpc):
                pltpu.make_async_copy(tbl_hbm.at[pl.ds(0, K)],
                                      rows.at[pl.ds(j * K, K)], gsem).wait()
                pltpu.make_async_copy(dtbl_hbm.at[pl.ds(0, K)],
                                      drows.at[pl.ds(j * K, K)], gsem).wait()

        def s_start(g, rows):
            for j in range(jpc):
                r = g * jpc + j
                pltpu.async_copy(rows.at[pl.ds(j * K, K)],
                                 acc.at[didx.at[r]], ssem, add=True)

        def s_drain(rows):
            for j in range(jpc):
                pltpu.make_async_copy(rows.at[pl.ds(j * K, K)],
                                      acc.at[pl.ds(0, K)], ssem).wait()

        def compute(rows, drows):
            @plsc.parallel_loop(0, sup, unroll=2)
            def _edge(e):
                dv = drows[e, pl.ds(0, 16)]
                if rw == RW1:
                    av = rows[e, pl.ds(64, 16)]
                    s = av + dv
                    w = jnp.exp(jnp.maximum(s, 0.2 * s))
                    for cc in range(4):
                        xwc = rows[e, pl.ds(cc * 16, 16)]
                        wb = w.at[pats[cc]].get(mode="promise_in_bounds")
                        rows[e, pl.ds(cc * 16, 16)] = xwc * wb
                    rows[e, pl.ds(64, 16)] = w
                else:
                    row = rows[e, pl.ds(0, 16)]
                    sv = row.at[splat8].get(mode="promise_in_bounds") + dv
                    w = jnp.exp(jnp.maximum(sv, 0.2 * sv))
                    out = jnp.where(m_lt8, w * row, jnp.where(m_eq8, w, zero_v))
                    rows[e, pl.ds(0, 16)] = out

        g_fire(0, rows_a, drows_a)

        @pl.loop(0, ch // 2)
        def _pair(t):
            g0 = 2 * t
            g1 = g0 + 1

            @pl.when(t > 0)
            def _():
                s_drain(rows_b)

            g_fire(g1, rows_b, drows_b)
            g_drain(rows_a, drows_a)
            compute(rows_a, drows_a)
            s_start(g0, rows_a)
            g_drain(rows_b, drows_b)
            compute(rows_b, drows_b)
            s_drain(rows_a)

            @pl.when(g1 + 1 < ch)
            def _():
                g_fire(g1 + 1, rows_a, drows_a)

            s_start(g1, rows_b)

        s_drain(rows_b)
        plsc.subcore_barrier()

        @pl.when((sid == 0) & (cid == 0))
        def _writeout0():
            pltpu.sync_copy(acc, out_a_hbm)

        @pl.when((sid == 0) & (cid == 1))
        def _writeout1():
            pltpu.sync_copy(acc, out_b_hbm)

    return edge_pass


def _tc1(xp, w1, a1s, a1d):
    blk = 1024

    def body(x_ref, w_ref, s_ref, d_ref, t_ref, dd_ref):
        xw = jnp.dot(x_ref[...], w_ref[...], preferred_element_type=jnp.float32)
        asrc = jnp.dot(xw, s_ref[...], preferred_element_type=jnp.float32)
        adst = jnp.dot(xw, d_ref[...], preferred_element_type=jnp.float32)
        z8 = jnp.zeros((blk, 8), jnp.float32)
        t_ref[...] = jnp.concatenate([xw, asrc, z8], axis=1)
        dd_ref[...] = jnp.concatenate([adst, z8], axis=1)

    return pl.pallas_call(
        body,
        grid=(NP // blk,),
        in_specs=[
            pl.BlockSpec((blk, F_IN), lambda i: (i, 0)),
            pl.BlockSpec((F_IN, 64), lambda i: (0, 0)),
            pl.BlockSpec((64, 8), lambda i: (0, 0)),
            pl.BlockSpec((64, 8), lambda i: (0, 0)),
        ],
        out_specs=[
            pl.BlockSpec((blk, RW1), lambda i: (i, 0)),
            pl.BlockSpec((blk, DW), lambda i: (i, 0)),
        ],
        out_shape=[
            jax.ShapeDtypeStruct((NP, RW1), jnp.float32),
            jax.ShapeDtypeStruct((NP, DW), jnp.float32),
        ],
    )(xp, w1, a1s, a1d)


def _tc2(acc1a, acc1b, b1, w2, as2, ad2):
    blk = 1024

    def body(aa_ref, ab_ref, b_ref, w_ref, s_ref, d_ref, t_ref, dd_ref):
        m = aa_ref[...] + ab_ref[...]
        msg = m[:, 0:64]
        den = m[:, 64:72]
        dex = jnp.broadcast_to(den[:, :, None], (blk, 8, 8)).reshape(blk, 64)
        h = msg / (dex + 1e-16) + b_ref[...]
        h = jnp.where(h > 0, h, jnp.exp(h) - 1.0)
        xw2 = jnp.dot(h, w_ref[...], preferred_element_type=jnp.float32)
        asrc2 = jnp.sum(xw2 * s_ref[...], axis=1, keepdims=True)
        adst2 = jnp.sum(xw2 * d_ref[...], axis=1, keepdims=True)
        z7 = jnp.zeros((blk, 7), jnp.float32)
        t_ref[...] = jnp.concatenate([xw2, asrc2, z7], axis=1)
        dd_ref[...] = jnp.broadcast_to(adst2, (blk, DW))

    return pl.pallas_call(
        body,
        grid=(NP // blk,),
        in_specs=[
            pl.BlockSpec((blk, RW1), lambda i: (i, 0)),
            pl.BlockSpec((blk, RW1), lambda i: (i, 0)),
            pl.BlockSpec((1, 64), lambda i: (0, 0)),
            pl.BlockSpec((64, 8), lambda i: (0, 0)),
            pl.BlockSpec((1, 8), lambda i: (0, 0)),
            pl.BlockSpec((1, 8), lambda i: (0, 0)),
        ],
        out_specs=[
            pl.BlockSpec((blk, RW2), lambda i: (i, 0)),
            pl.BlockSpec((blk, DW), lambda i: (i, 0)),
        ],
        out_shape=[
            jax.ShapeDtypeStruct((NP, RW2), jnp.float32),
            jax.ShapeDtypeStruct((NP, DW), jnp.float32),
        ],
    )(acc1a, acc1b, b1, w2, as2, ad2)


def _tc3(acc2a, acc2b, b2):
    blk = 2000

    def body(aa_ref, ab_ref, b_ref, o_ref):
        m = aa_ref[...] + ab_ref[...]
        v = m[:, 0:8] / (m[:, 8:9] + 1e-16) + b_ref[...]
        mx = jnp.max(v, axis=1, keepdims=True)
        lse = mx + jnp.log(jnp.sum(jnp.exp(v - mx), axis=1, keepdims=True))
        o_ref[...] = v - lse

    return pl.pallas_call(
        body,
        grid=(N // blk,),
        in_specs=[
            pl.BlockSpec((blk, RW2), lambda i: (i, 0)),
            pl.BlockSpec((blk, RW2), lambda i: (i, 0)),
            pl.BlockSpec((1, 8), lambda i: (0, 0)),
        ],
        out_specs=pl.BlockSpec((blk, 8), lambda i: (i, 0)),
        out_shape=jax.ShapeDtypeStruct((N, 8), jnp.float32),
    )(acc2a, acc2b, b2)


def kernel(x, edge_index, W1, att_src1, att_dst1, b1,
           W2, att_src2, att_dst2, b2):
    e = edge_index.shape[1]
    tot = e + N
    jpc1 = jpc2 = 2
    ch1 = ch2 = 2 * -(-tot // (2 * NTILES * K * jpc1))
    ep = NTILES * K * jpc1 * ch1

    loops = jnp.arange(N, dtype=jnp.int32)
    pad = N + jnp.arange(ep - tot, dtype=jnp.int32) % (NP - N)
    src = jnp.concatenate([edge_index[0], loops, pad]).reshape(ep // K, K)
    dst = jnp.concatenate([edge_index[1], loops, pad]).reshape(ep // K, K)

    xp = jnp.pad(x, ((0, NP - N), (0, 0)))
    eye8 = jnp.eye(8, dtype=jnp.float32)
    a1s = (att_src1[:, :, None] * eye8[:, None, :]).reshape(64, 8)
    a1d = (att_dst1[:, :, None] * eye8[:, None, :]).reshape(64, 8)

    t1, d1 = _tc1(xp, W1, a1s, a1d)
    zero1 = jnp.zeros((NP, RW1), jnp.float32)
    acc1a, acc1b = _make_edge_pass(ch1, NP, RW1, jpc1)(src, dst, t1, d1, zero1)

    t2, d2 = _tc2(acc1a, acc1b, b1.reshape(1, 64), W2,
                  att_src2.reshape(1, 8), att_dst2.reshape(1, 8))
    zero2 = jnp.zeros((NP, RW2), jnp.float32)
    acc2a, acc2b = _make_edge_pass(ch2, NP, RW2, jpc2)(src, dst, t2, d2, zero2)

    return _tc3(acc2a, acc2b, b2.reshape(1, 8))


# speedup vs baseline: 183.2181x; 1.0566x over previous
import dataclasses
import functools

import jax
import jax.numpy as jnp
from jax import lax
from jax.experimental import pallas as pl
from jax.experimental.pallas import tpu as pltpu
from jax.experimental.pallas import tpu_sc as plsc

N = 10000
NP = 10240
F_IN = 128
RW1 = 80
RW2 = 16
DW = 16
K = 128
NTILES = 32


def _make_edge_pass(ch, nrow, rw, jpc):
    sup = K * jpc
    mesh = plsc.VectorSubcoreMesh(core_axis_name="c", subcore_axis_name="s")
    cp = pltpu.CompilerParams()
    if "needs_layout_passes" in pltpu.CompilerParams.__dataclass_fields__:
        cp = dataclasses.replace(cp, needs_layout_passes=False)
    if "use_tc_tiling_on_sc" in pltpu.CompilerParams.__dataclass_fields__:
        cp = dataclasses.replace(cp, use_tc_tiling_on_sc=False)

    nidx = ch * jpc

    @functools.partial(
        pl.kernel,
        out_type=[jax.ShapeDtypeStruct((nrow, rw), jnp.float32),
                  jax.ShapeDtypeStruct((nrow, rw), jnp.float32)],
        mesh=mesh,
        compiler_params=cp,
        scratch_types=[
            pltpu.VMEM_SHARED((nrow, rw), jnp.float32),
            pltpu.VMEM((nidx, K), jnp.int32),
            pltpu.VMEM((nidx, K), jnp.int32),
            pltpu.VMEM((sup, rw), jnp.float32),
            pltpu.VMEM((sup, rw), jnp.float32),
            pltpu.VMEM((sup, DW), jnp.float32),
            pltpu.VMEM((sup, DW), jnp.float32),
            pltpu.SemaphoreType.DMA,
            pltpu.SemaphoreType.DMA,
        ],
    )
    def edge_pass(src_hbm, dst_hbm, tbl_hbm, dtbl_hbm, zero_hbm,
                  out_a_hbm, out_b_hbm,
                  acc, sidx, didx, rows_a, rows_b, drows_a, drows_b,
                  gsem, ssem):
        cid = lax.axis_index("c")
        sid = lax.axis_index("s")
        wid = cid * 16 + sid

        zrows = nrow // 16
        pltpu.sync_copy(zero_hbm.at[pl.ds(sid * zrows, zrows)],
                        acc.at[pl.ds(sid * zrows, zrows)])
        pltpu.sync_copy(src_hbm.at[pl.ds(wid * nidx, nidx)], sidx)
        pltpu.sync_copy(dst_hbm.at[pl.ds(wid * nidx, nidx)], didx)
        plsc.subcore_barrier()

        iota = lax.iota(jnp.int32, 16)
        if rw == RW1:
            pats = [iota // 8 + 2 * cc for cc in range(4)]
        else:
            splat8 = jnp.zeros((16,), jnp.int32) + 8
            m_lt8 = iota < 8
            m_eq8 = iota == 8
            zero_v = jnp.zeros((16,), jnp.float32)

        def g_fire(g, rows, drows):
            for j in range(jpc):
                r = g * jpc + j
                pltpu.async_copy(tbl_hbm.at[sidx.at[r]],
                                 rows.at[pl.ds(j * K, K)], gsem)
                pltpu.async_copy(dtbl_hbm.at[didx.at[r]],
                                 drows.at[pl.ds(j * K, K)], gsem)

        def g_drain(rows, drows):
            for j in range(jpc):
                pltpu.make_async_copy(tbl_hbm.at[pl.ds(0, K)],
                                      rows.at[pl.ds(j * K, K)], gsem).wait()
                pltpu.make_async_copy(dtbl_hbm.at[pl.ds(0, K)],
                                      drows.at[pl.ds(j * K, K)], gsem).wait()

        def s_start(g, rows):
            for j in range(jpc):
                r = g * jpc + j
                pltpu.async_copy(rows.at[pl.ds(j * K, K)],
                                 acc.at[didx.at[r]], ssem, add=True)

        def s_drain(rows):
            for j in range(jpc):
                pltpu.make_async_copy(rows.at[pl.ds(j * K, K)],
                                      acc.at[pl.ds(0, K)], ssem).wait()

        def compute(rows, drows):
            @plsc.parallel_loop(0, sup, unroll=4)
            def _edge(e):
                dv = drows[e, pl.ds(0, 16)]
                if rw == RW1:
                    av = rows[e, pl.ds(64, 16)]
                    s = av + dv
                    w = jnp.exp(jnp.maximum(s, 0.2 * s))
                    for cc in range(4):
                        xwc = rows[e, pl.ds(cc * 16, 16)]
                        wb = w.at[pats[cc]].get(mode="promise_in_bounds")
                        rows[e, pl.ds(cc * 16, 16)] = xwc * wb
                    rows[e, pl.ds(64, 16)] = w
                else:
                    row = rows[e, pl.ds(0, 16)]
                    sv = row.at[splat8].get(mode="promise_in_bounds") + dv
                    w = jnp.exp(jnp.maximum(sv, 0.2 * sv))
                    out = jnp.where(m_lt8, w * row, jnp.where(m_eq8, w, zero_v))
                    rows[e, pl.ds(0, 16)] = out

        g_fire(0, rows_a, drows_a)

        @pl.loop(0, ch // 2)
        def _pair(t):
            g0 = 2 * t
            g1 = g0 + 1

            @pl.when(t > 0)
            def _():
                s_drain(rows_b)

            g_fire(g1, rows_b, drows_b)
            g_drain(rows_a, drows_a)
            compute(rows_a, drows_a)
            s_start(g0, rows_a)
            g_drain(rows_b, drows_b)
            compute(rows_b, drows_b)
            s_drain(rows_a)

            @pl.when(g1 + 1 < ch)
            def _():
                g_fire(g1 + 1, rows_a, drows_a)

            s_start(g1, rows_b)

        s_drain(rows_b)
        plsc.subcore_barrier()

        @pl.when((sid == 0) & (cid == 0))
        def _writeout0():
            pltpu.sync_copy(acc, out_a_hbm)

        @pl.when((sid == 0) & (cid == 1))
        def _writeout1():
            pltpu.sync_copy(acc, out_b_hbm)

    return edge_pass


def _tc1(xp, w1, a1s, a1d):
    blk = 1024

    def body(x_ref, w_ref, s_ref, d_ref, t_ref, dd_ref):
        xw = jnp.dot(x_ref[...], w_ref[...], preferred_element_type=jnp.float32)
        asrc = jnp.dot(xw, s_ref[...], preferred_element_type=jnp.float32)
        adst = jnp.dot(xw, d_ref[...], preferred_element_type=jnp.float32)
        z8 = jnp.zeros((blk, 8), jnp.float32)
        t_ref[...] = jnp.concatenate([xw, asrc, z8], axis=1)
        dd_ref[...] = jnp.concatenate([adst, z8], axis=1)

    return pl.pallas_call(
        body,
        grid=(NP // blk,),
        in_specs=[
            pl.BlockSpec((blk, F_IN), lambda i: (i, 0)),
            pl.BlockSpec((F_IN, 64), lambda i: (0, 0)),
            pl.BlockSpec((64, 8), lambda i: (0, 0)),
            pl.BlockSpec((64, 8), lambda i: (0, 0)),
        ],
        out_specs=[
            pl.BlockSpec((blk, RW1), lambda i: (i, 0)),
            pl.BlockSpec((blk, DW), lambda i: (i, 0)),
        ],
        out_shape=[
            jax.ShapeDtypeStruct((NP, RW1), jnp.float32),
            jax.ShapeDtypeStruct((NP, DW), jnp.float32),
        ],
    )(xp, w1, a1s, a1d)


def _tc2(acc1a, acc1b, rep8, b1, w2, as2, ad2):
    blk = 1024

    def body(aa_ref, ab_ref, rep_ref, b_ref, w_ref, s_ref, d_ref, t_ref,
             dd_ref):
        m = aa_ref[...] + ab_ref[...]
        msg = m[:, 0:64]
        den = m[:, 64:72]
        dex = jnp.dot(den, rep_ref[...], preferred_element_type=jnp.float32)
        h = msg / (dex + 1e-16) + b_ref[...]
        h = jnp.where(h > 0, h, jnp.exp(h) - 1.0)
        xw2 = jnp.dot(h, w_ref[...], preferred_element_type=jnp.float32)
        asrc2 = jnp.sum(xw2 * s_ref[...], axis=1, keepdims=True)
        adst2 = jnp.sum(xw2 * d_ref[...], axis=1, keepdims=True)
        z7 = jnp.zeros((blk, 7), jnp.float32)
        t_ref[...] = jnp.concatenate([xw2, asrc2, z7], axis=1)
        dd_ref[...] = jnp.broadcast_to(adst2, (blk, DW))

    return pl.pallas_call(
        body,
        grid=(NP // blk,),
        in_specs=[
            pl.BlockSpec((blk, RW1), lambda i: (i, 0)),
            pl.BlockSpec((blk, RW1), lambda i: (i, 0)),
            pl.BlockSpec((8, 64), lambda i: (0, 0)),
            pl.BlockSpec((1, 64), lambda i: (0, 0)),
            pl.BlockSpec((64, 8), lambda i: (0, 0)),
            pl.BlockSpec((1, 8), lambda i: (0, 0)),
            pl.BlockSpec((1, 8), lambda i: (0, 0)),
        ],
        out_specs=[
            pl.BlockSpec((blk, RW2), lambda i: (i, 0)),
            pl.BlockSpec((blk, DW), lambda i: (i, 0)),
        ],
        out_shape=[
            jax.ShapeDtypeStruct((NP, RW2), jnp.float32),
            jax.ShapeDtypeStruct((NP, DW), jnp.float32),
        ],
    )(acc1a, acc1b, rep8, b1, w2, as2, ad2)


def _tc3(acc2a, acc2b, b2):
    blk = 2000

    def body(aa_ref, ab_ref, b_ref, o_ref):
        m = aa_ref[...] + ab_ref[...]
        v = m[:, 0:8] / (m[:, 8:9] + 1e-16) + b_ref[...]
        mx = jnp.max(v, axis=1, keepdims=True)
        lse = mx + jnp.log(jnp.sum(jnp.exp(v - mx), axis=1, keepdims=True))
        o_ref[...] = v - lse

    return pl.pallas_call(
        body,
        grid=(N // blk,),
        in_specs=[
            pl.BlockSpec((blk, RW2), lambda i: (i, 0)),
            pl.BlockSpec((blk, RW2), lambda i: (i, 0)),
            pl.BlockSpec((1, 8), lambda i: (0, 0)),
        ],
        out_specs=pl.BlockSpec((blk, 8), lambda i: (i, 0)),
        out_shape=jax.ShapeDtypeStruct((N, 8), jnp.float32),
    )(acc2a, acc2b, b2)


def kernel(x, edge_index, W1, att_src1, att_dst1, b1,
           W2, att_src2, att_dst2, b2):
    e = edge_index.shape[1]
    tot = e + N
    jpc1 = jpc2 = 2
    ch1 = ch2 = 2 * -(-tot // (2 * NTILES * K * jpc1))
    ep = NTILES * K * jpc1 * ch1

    loops = jnp.arange(N, dtype=jnp.int32)
    pad = N + jnp.arange(ep - tot, dtype=jnp.int32) % (NP - N)
    src = jnp.concatenate([edge_index[0], loops, pad]).reshape(ep // K, K)
    dst = jnp.concatenate([edge_index[1], loops, pad]).reshape(ep // K, K)

    xp = jnp.pad(x, ((0, NP - N), (0, 0)))
    eye8 = jnp.eye(8, dtype=jnp.float32)
    a1s = (att_src1[:, :, None] * eye8[:, None, :]).reshape(64, 8)
    a1d = (att_dst1[:, :, None] * eye8[:, None, :]).reshape(64, 8)

    t1, d1 = _tc1(xp, W1, a1s, a1d)
    zero1 = jnp.zeros((NP, RW1), jnp.float32)
    acc1a, acc1b = _make_edge_pass(ch1, NP, RW1, jpc1)(src, dst, t1, d1, zero1)

    rep8 = jnp.repeat(eye8, 8, axis=1)
    t2, d2 = _tc2(acc1a, acc1b, rep8, b1.reshape(1, 64), W2,
                  att_src2.reshape(1, 8), att_dst2.reshape(1, 8))
    zero2 = jnp.zeros((NP, RW2), jnp.float32)
    acc2a, acc2b = _make_edge_pass(ch2, NP, RW2, jpc2)(src, dst, t2, d2, zero2)

    return _tc3(acc2a, acc2b, b2.reshape(1, 8))


# speedup vs baseline: 184.8627x; 1.0090x over previous
import dataclasses
import functools

import jax
import jax.numpy as jnp
from jax import lax
from jax.experimental import pallas as pl
from jax.experimental.pallas import tpu as pltpu
from jax.experimental.pallas import tpu_sc as plsc

N = 10000
NP = 10240
F_IN = 128
RW1 = 72
RW2 = 16
DW = 16
K = 128
NTILES = 32


def _make_edge_pass(ch, nrow, rw, jpc):
    sup = K * jpc
    mesh = plsc.VectorSubcoreMesh(core_axis_name="c", subcore_axis_name="s")
    cp = pltpu.CompilerParams()
    if "needs_layout_passes" in pltpu.CompilerParams.__dataclass_fields__:
        cp = dataclasses.replace(cp, needs_layout_passes=False)
    if "use_tc_tiling_on_sc" in pltpu.CompilerParams.__dataclass_fields__:
        cp = dataclasses.replace(cp, use_tc_tiling_on_sc=False)

    nidx = ch * jpc

    @functools.partial(
        pl.kernel,
        out_type=[jax.ShapeDtypeStruct((nrow, rw), jnp.float32),
                  jax.ShapeDtypeStruct((nrow, rw), jnp.float32)],
        mesh=mesh,
        compiler_params=cp,
        scratch_types=[
            pltpu.VMEM_SHARED((nrow, rw), jnp.float32),
            pltpu.VMEM((nidx, K), jnp.int32),
            pltpu.VMEM((nidx, K), jnp.int32),
            pltpu.VMEM((sup, rw), jnp.float32),
            pltpu.VMEM((sup, rw), jnp.float32),
            pltpu.VMEM((sup, DW), jnp.float32),
            pltpu.VMEM((sup, DW), jnp.float32),
            pltpu.SemaphoreType.DMA,
            pltpu.SemaphoreType.DMA,
        ],
    )
    def edge_pass(src_hbm, dst_hbm, tbl_hbm, dtbl_hbm, zero_hbm,
                  out_a_hbm, out_b_hbm,
                  acc, sidx, didx, rows_a, rows_b, drows_a, drows_b,
                  gsem, ssem):
        cid = lax.axis_index("c")
        sid = lax.axis_index("s")
        wid = cid * 16 + sid

        zrows = nrow // 16
        pltpu.sync_copy(zero_hbm.at[pl.ds(sid * zrows, zrows)],
                        acc.at[pl.ds(sid * zrows, zrows)])
        pltpu.sync_copy(src_hbm.at[pl.ds(wid * nidx, nidx)], sidx)
        pltpu.sync_copy(dst_hbm.at[pl.ds(wid * nidx, nidx)], didx)
        plsc.subcore_barrier()

        iota = lax.iota(jnp.int32, 16)
        if rw == RW1:
            pats = [8 + iota // 8 + 2 * cc for cc in range(4)]
            pat_tail = (iota + 8) & 15
            m_lt8 = iota < 8
        else:
            splat8 = jnp.zeros((16,), jnp.int32) + 8
            m_lt8 = iota < 8
            m_eq8 = iota == 8
            zero_v = jnp.zeros((16,), jnp.float32)

        def g_fire(g, rows, drows):
            for j in range(jpc):
                r = g * jpc + j
                pltpu.async_copy(tbl_hbm.at[sidx.at[r]],
                                 rows.at[pl.ds(j * K, K)], gsem)
                pltpu.async_copy(dtbl_hbm.at[didx.at[r]],
                                 drows.at[pl.ds(j * K, K)], gsem)

        def g_drain(rows, drows):
            for j in range(jpc):
                pltpu.make_async_copy(tbl_hbm.at[pl.ds(0, K)],
                                      rows.at[pl.ds(j * K, K)], gsem).wait()
                pltpu.make_async_copy(dtbl_hbm.at[pl.ds(0, K)],
                                      drows.at[pl.ds(j * K, K)], gsem).wait()

        def s_start(g, rows):
            for j in range(jpc):
                r = g * jpc + j
                pltpu.async_copy(rows.at[pl.ds(j * K, K)],
                                 acc.at[didx.at[r]], ssem, add=True)

        def s_drain(rows):
            for j in range(jpc):
                pltpu.make_async_copy(rows.at[pl.ds(j * K, K)],
                                      acc.at[pl.ds(0, K)], ssem).wait()

        def compute(rows, drows):
            @plsc.parallel_loop(0, sup, unroll=4)
            def _edge(e):
                dv = drows[e, pl.ds(0, 16)]
                if rw == RW1:
                    av = rows[e, pl.ds(56, 16)]
                    s = av + dv
                    w = jnp.exp(jnp.maximum(s, 0.2 * s))
                    m3 = None
                    for cc in range(4):
                        xwc = rows[e, pl.ds(cc * 16, 16)]
                        wb = w.at[pats[cc]].get(mode="promise_in_bounds")
                        m3 = xwc * wb
                        rows[e, pl.ds(cc * 16, 16)] = m3
                    m3s = m3.at[pat_tail].get(mode="promise_in_bounds")
                    rows[e, pl.ds(56, 16)] = jnp.where(m_lt8, m3s, w)
                else:
                    row = rows[e, pl.ds(0, 16)]
                    sv = row.at[splat8].get(mode="promise_in_bounds") + dv
                    w = jnp.exp(jnp.maximum(sv, 0.2 * sv))
                    out = jnp.where(m_lt8, w * row, jnp.where(m_eq8, w, zero_v))
                    rows[e, pl.ds(0, 16)] = out

        g_fire(0, rows_a, drows_a)

        @pl.loop(0, ch // 2)
        def _pair(t):
            g0 = 2 * t
            g1 = g0 + 1

            @pl.when(t > 0)
            def _():
                s_drain(rows_b)

            g_fire(g1, rows_b, drows_b)
            g_drain(rows_a, drows_a)
            compute(rows_a, drows_a)
            s_start(g0, rows_a)
            g_drain(rows_b, drows_b)
            compute(rows_b, drows_b)
            s_drain(rows_a)

            @pl.when(g1 + 1 < ch)
            def _():
                g_fire(g1 + 1, rows_a, drows_a)

            s_start(g1, rows_b)

        s_drain(rows_b)
        plsc.subcore_barrier()

        @pl.when((sid == 0) & (cid == 0))
        def _writeout0():
            pltpu.sync_copy(acc, out_a_hbm)

        @pl.when((sid == 0) & (cid == 1))
        def _writeout1():
            pltpu.sync_copy(acc, out_b_hbm)

    return edge_pass


def _tc1(xp, w1, a1s, a1d):
    blk = 1024

    def body(x_ref, w_ref, s_ref, d_ref, t_ref, dd_ref):
        xw = jnp.dot(x_ref[...], w_ref[...], preferred_element_type=jnp.float32)
        asrc = jnp.dot(xw, s_ref[...], preferred_element_type=jnp.float32)
        adst = jnp.dot(xw, d_ref[...], preferred_element_type=jnp.float32)
        z8 = jnp.zeros((blk, 8), jnp.float32)
        t_ref[...] = jnp.concatenate([xw, asrc], axis=1)
        dd_ref[...] = jnp.concatenate([z8, adst], axis=1)

    return pl.pallas_call(
        body,
        grid=(NP // blk,),
        in_specs=[
            pl.BlockSpec((blk, F_IN), lambda i: (i, 0)),
            pl.BlockSpec((F_IN, 64), lambda i: (0, 0)),
            pl.BlockSpec((64, 8), lambda i: (0, 0)),
            pl.BlockSpec((64, 8), lambda i: (0, 0)),
        ],
        out_specs=[
            pl.BlockSpec((blk, RW1), lambda i: (i, 0)),
            pl.BlockSpec((blk, DW), lambda i: (i, 0)),
        ],
        out_shape=[
            jax.ShapeDtypeStruct((NP, RW1), jnp.float32),
            jax.ShapeDtypeStruct((NP, DW), jnp.float32),
        ],
    )(xp, w1, a1s, a1d)


def _tc2(acc1a, acc1b, rep8, b1, w2, as2, ad2):
    blk = 1024

    def body(aa_ref, ab_ref, rep_ref, b_ref, w_ref, s_ref, d_ref, t_ref,
             dd_ref):
        m = aa_ref[...] + ab_ref[...]
        msg = m[:, 0:64]
        den = m[:, 64:72]
        dex = jnp.dot(den, rep_ref[...], preferred_element_type=jnp.float32)
        h = msg / (dex + 1e-16) + b_ref[...]
        h = jnp.where(h > 0, h, jnp.exp(h) - 1.0)
        xw2 = jnp.dot(h, w_ref[...], preferred_element_type=jnp.float32)
        asrc2 = jnp.sum(xw2 * s_ref[...], axis=1, keepdims=True)
        adst2 = jnp.sum(xw2 * d_ref[...], axis=1, keepdims=True)
        z7 = jnp.zeros((blk, 7), jnp.float32)
        t_ref[...] = jnp.concatenate([xw2, asrc2, z7], axis=1)
        dd_ref[...] = jnp.broadcast_to(adst2, (blk, DW))

    return pl.pallas_call(
        body,
        grid=(NP // blk,),
        in_specs=[
            pl.BlockSpec((blk, RW1), lambda i: (i, 0)),
            pl.BlockSpec((blk, RW1), lambda i: (i, 0)),
            pl.BlockSpec((8, 64), lambda i: (0, 0)),
            pl.BlockSpec((1, 64), lambda i: (0, 0)),
            pl.BlockSpec((64, 8), lambda i: (0, 0)),
            pl.BlockSpec((1, 8), lambda i: (0, 0)),
            pl.BlockSpec((1, 8), lambda i: (0, 0)),
        ],
        out_specs=[
            pl.BlockSpec((blk, RW2), lambda i: (i, 0)),
            pl.BlockSpec((blk, DW), lambda i: (i, 0)),
        ],
        out_shape=[
            jax.ShapeDtypeStruct((NP, RW2), jnp.float32),
            jax.ShapeDtypeStruct((NP, DW), jnp.float32),
        ],
    )(acc1a, acc1b, rep8, b1, w2, as2, ad2)


def _tc3(acc2a, acc2b, b2):
    blk = 2000

    def body(aa_ref, ab_ref, b_ref, o_ref):
        m = aa_ref[...] + ab_ref[...]
        v = m[:, 0:8] / (m[:, 8:9] + 1e-16) + b_ref[...]
        mx = jnp.max(v, axis=1, keepdims=True)
        lse = mx + jnp.log(jnp.sum(jnp.exp(v - mx), axis=1, keepdims=True))
        o_ref[...] = v - lse

    return pl.pallas_call(
        body,
        grid=(N // blk,),
        in_specs=[
            pl.BlockSpec((blk, RW2), lambda i: (i, 0)),
            pl.BlockSpec((blk, RW2), lambda i: (i, 0)),
            pl.BlockSpec((1, 8), lambda i: (0, 0)),
        ],
        out_specs=pl.BlockSpec((blk, 8), lambda i: (i, 0)),
        out_shape=jax.ShapeDtypeStruct((N, 8), jnp.float32),
    )(acc2a, acc2b, b2)


def kernel(x, edge_index, W1, att_src1, att_dst1, b1,
           W2, att_src2, att_dst2, b2):
    e = edge_index.shape[1]
    tot = e + N
    jpc1, jpc2 = 2, 4
    ch1 = 2 * -(-tot // (2 * NTILES * K * jpc1))
    ch2 = 2 * -(-tot // (2 * NTILES * K * jpc2))
    ep = NTILES * K * max(jpc1 * ch1, jpc2 * ch2)

    loops = jnp.arange(N, dtype=jnp.int32)
    pad = N + jnp.arange(ep - tot, dtype=jnp.int32) % (NP - N)
    src = jnp.concatenate([edge_index[0], loops, pad]).reshape(ep // K, K)
    dst = jnp.concatenate([edge_index[1], loops, pad]).reshape(ep // K, K)

    xp = jnp.pad(x, ((0, NP - N), (0, 0)))
    eye8 = jnp.eye(8, dtype=jnp.float32)
    a1s = (att_src1[:, :, None] * eye8[:, None, :]).reshape(64, 8)
    a1d = (att_dst1[:, :, None] * eye8[:, None, :]).reshape(64, 8)

    t1, d1 = _tc1(xp, W1, a1s, a1d)
    zero1 = jnp.zeros((NP, RW1), jnp.float32)
    acc1a, acc1b = _make_edge_pass(ch1, NP, RW1, jpc1)(src, dst, t1, d1, zero1)

    rep8 = jnp.repeat(eye8, 8, axis=1)
    t2, d2 = _tc2(acc1a, acc1b, rep8, b1.reshape(1, 64), W2,
                  att_src2.reshape(1, 8), att_dst2.reshape(1, 8))
    zero2 = jnp.zeros((NP, RW2), jnp.float32)
    acc2a, acc2b = _make_edge_pass(ch2, NP, RW2, jpc2)(src, dst, t2, d2, zero2)

    return _tc3(acc2a, acc2b, b2.reshape(1, 8))


# speedup vs baseline: 203.2055x; 1.0992x over previous
import dataclasses
import functools

import jax
import jax.numpy as jnp
from jax import lax
from jax.experimental import pallas as pl
from jax.experimental.pallas import tpu as pltpu
from jax.experimental.pallas import tpu_sc as plsc

N = 10000
NP = 10240
F_IN = 128
RW1 = 72
RW2 = 16
DW = 16
K = 128
NTILES = 32


def _make_edge_pass(ch, nrow, rw, jpc, er, tot):
    sup = K * jpc
    mesh = plsc.VectorSubcoreMesh(core_axis_name="c", subcore_axis_name="s")
    cp = pltpu.CompilerParams()
    if "needs_layout_passes" in pltpu.CompilerParams.__dataclass_fields__:
        cp = dataclasses.replace(cp, needs_layout_passes=False)
    if "use_tc_tiling_on_sc" in pltpu.CompilerParams.__dataclass_fields__:
        cp = dataclasses.replace(cp, use_tc_tiling_on_sc=False)

    nidx = ch * jpc

    @functools.partial(
        pl.kernel,
        out_type=[jax.ShapeDtypeStruct((nrow, rw), jnp.float32),
                  jax.ShapeDtypeStruct((nrow, rw), jnp.float32)],
        mesh=mesh,
        compiler_params=cp,
        scratch_types=[
            pltpu.VMEM_SHARED((nrow, rw), jnp.float32),
            pltpu.VMEM((nidx, K), jnp.int32),
            pltpu.VMEM((nidx, K), jnp.int32),
            pltpu.VMEM((sup, rw), jnp.float32),
            pltpu.VMEM((sup, rw), jnp.float32),
            pltpu.VMEM((sup, DW), jnp.float32),
            pltpu.VMEM((sup, DW), jnp.float32),
            pltpu.SemaphoreType.DMA,
            pltpu.SemaphoreType.DMA,
            pltpu.SemaphoreType.DMA,
        ],
    )
    def edge_pass(ei_hbm, tbl_hbm, dtbl_hbm, zero_hbm,
                  out_a_hbm, out_b_hbm,
                  acc, sidx, didx, rows_a, rows_b, drows_a, drows_b,
                  gsem, ssem, isem):
        cid = lax.axis_index("c")
        sid = lax.axis_index("s")
        wid = cid * 16 + sid
        iota = lax.iota(jnp.int32, 16)

        zrows = nrow // 16
        pltpu.sync_copy(zero_hbm.at[pl.ds(sid * zrows, zrows)],
                        acc.at[pl.ds(sid * zrows, zrows)])

        @pl.loop(0, nidx)
        def _fill(i):
            r = wid * nidx + i

            @pl.when(r < er)
            def _():
                pltpu.async_copy(ei_hbm.at[0, r], sidx.at[i], isem)
                pltpu.async_copy(ei_hbm.at[1, r], didx.at[i], isem)

            @pl.when(r >= er)
            def _():
                for j in range(8):
                    p = r * K + 16 * j + iota
                    v = jnp.where(p < tot, p - er * K,
                                  N + lax.rem(p, nrow - N))
                    sidx[i, pl.ds(16 * j, 16)] = v
                    didx[i, pl.ds(16 * j, 16)] = v

        @pl.loop(0, nidx)
        def _dr(i):
            r = wid * nidx + i

            @pl.when(r < er)
            def _():
                pltpu.make_async_copy(ei_hbm.at[0, r], sidx.at[i], isem).wait()
                pltpu.make_async_copy(ei_hbm.at[1, r], didx.at[i], isem).wait()

        plsc.subcore_barrier()
        if rw == RW1:
            pats = [8 + iota // 8 + 2 * cc for cc in range(4)]
            pat_tail = (iota + 8) & 15
            m_lt8 = iota < 8
        else:
            splat8 = jnp.zeros((16,), jnp.int32) + 8
            m_lt8 = iota < 8
            m_eq8 = iota == 8
            zero_v = jnp.zeros((16,), jnp.float32)

        def g_fire(g, rows, drows):
            for j in range(jpc):
                r = g * jpc + j
                pltpu.async_copy(tbl_hbm.at[sidx.at[r]],
                                 rows.at[pl.ds(j * K, K)], gsem)
                pltpu.async_copy(dtbl_hbm.at[didx.at[r]],
                                 drows.at[pl.ds(j * K, K)], gsem)

        def g_drain(rows, drows):
            for j in range(jpc):
                pltpu.make_async_copy(tbl_hbm.at[pl.ds(0, K)],
                                      rows.at[pl.ds(j * K, K)], gsem).wait()
                pltpu.make_async_copy(dtbl_hbm.at[pl.ds(0, K)],
                                      drows.at[pl.ds(j * K, K)], gsem).wait()

        def s_start(g, rows):
            for j in range(jpc):
                r = g * jpc + j
                pltpu.async_copy(rows.at[pl.ds(j * K, K)],
                                 acc.at[didx.at[r]], ssem, add=True)

        def s_drain(rows):
            for j in range(jpc):
                pltpu.make_async_copy(rows.at[pl.ds(j * K, K)],
                                      acc.at[pl.ds(0, K)], ssem).wait()

        def compute(rows, drows):
            @plsc.parallel_loop(0, sup, unroll=4)
            def _edge(e):
                dv = drows[e, pl.ds(0, 16)]
                if rw == RW1:
                    av = rows[e, pl.ds(56, 16)]
                    s = av + dv
                    w = jnp.exp(jnp.maximum(s, 0.2 * s))
                    m3 = None
                    for cc in range(4):
                        xwc = rows[e, pl.ds(cc * 16, 16)]
                        wb = w.at[pats[cc]].get(mode="promise_in_bounds")
                        m3 = xwc * wb
                        rows[e, pl.ds(cc * 16, 16)] = m3
                    m3s = m3.at[pat_tail].get(mode="promise_in_bounds")
                    rows[e, pl.ds(56, 16)] = jnp.where(m_lt8, m3s, w)
                else:
                    row = rows[e, pl.ds(0, 16)]
                    sv = row.at[splat8].get(mode="promise_in_bounds") + dv
                    w = jnp.exp(jnp.maximum(sv, 0.2 * sv))
                    out = jnp.where(m_lt8, w * row, jnp.where(m_eq8, w, zero_v))
                    rows[e, pl.ds(0, 16)] = out

        g_fire(0, rows_a, drows_a)

        @pl.loop(0, ch // 2)
        def _pair(t):
            g0 = 2 * t
            g1 = g0 + 1

            @pl.when(t > 0)
            def _():
                s_drain(rows_b)

            g_fire(g1, rows_b, drows_b)
            g_drain(rows_a, drows_a)
            compute(rows_a, drows_a)
            s_start(g0, rows_a)
            g_drain(rows_b, drows_b)
            compute(rows_b, drows_b)
            s_drain(rows_a)

            @pl.when(g1 + 1 < ch)
            def _():
                g_fire(g1 + 1, rows_a, drows_a)

            s_start(g1, rows_b)

        s_drain(rows_b)
        plsc.subcore_barrier()

        @pl.when((sid == 0) & (cid == 0))
        def _writeout0():
            pltpu.sync_copy(acc, out_a_hbm)

        @pl.when((sid == 0) & (cid == 1))
        def _writeout1():
            pltpu.sync_copy(acc, out_b_hbm)

    return edge_pass


def _tc1(xp, w1, a1s, a1d):
    blk = 1024

    def body(x_ref, w_ref, s_ref, d_ref, t_ref, dd_ref):
        xw = jnp.dot(x_ref[...], w_ref[...], preferred_element_type=jnp.float32)
        asrc = jnp.dot(xw, s_ref[...], preferred_element_type=jnp.float32)
        adst = jnp.dot(xw, d_ref[...], preferred_element_type=jnp.float32)
        z8 = jnp.zeros((blk, 8), jnp.float32)
        t_ref[...] = jnp.concatenate([xw, asrc], axis=1)
        dd_ref[...] = jnp.concatenate([z8, adst], axis=1)

    return pl.pallas_call(
        body,
        grid=(NP // blk,),
        in_specs=[
            pl.BlockSpec((blk, F_IN), lambda i: (i, 0)),
            pl.BlockSpec((F_IN, 64), lambda i: (0, 0)),
            pl.BlockSpec((64, 8), lambda i: (0, 0)),
            pl.BlockSpec((64, 8), lambda i: (0, 0)),
        ],
        out_specs=[
            pl.BlockSpec((blk, RW1), lambda i: (i, 0)),
            pl.BlockSpec((blk, DW), lambda i: (i, 0)),
        ],
        out_shape=[
            jax.ShapeDtypeStruct((NP, RW1), jnp.float32),
            jax.ShapeDtypeStruct((NP, DW), jnp.float32),
        ],
    )(xp, w1, a1s, a1d)


def _tc2(acc1a, acc1b, rep8, b1, w2, as2, ad2):
    blk = 1024

    def body(aa_ref, ab_ref, rep_ref, b_ref, w_ref, s_ref, d_ref, t_ref,
             dd_ref):
        m = aa_ref[...] + ab_ref[...]
        msg = m[:, 0:64]
        den = m[:, 64:72]
        dex = jnp.dot(den, rep_ref[...], preferred_element_type=jnp.float32)
        h = msg / (dex + 1e-16) + b_ref[...]
        h = jnp.where(h > 0, h, jnp.exp(h) - 1.0)
        xw2 = jnp.dot(h, w_ref[...], preferred_element_type=jnp.float32)
        asrc2 = jnp.sum(xw2 * s_ref[...], axis=1, keepdims=True)
        adst2 = jnp.sum(xw2 * d_ref[...], axis=1, keepdims=True)
        z7 = jnp.zeros((blk, 7), jnp.float32)
        t_ref[...] = jnp.concatenate([xw2, asrc2, z7], axis=1)
        dd_ref[...] = jnp.broadcast_to(adst2, (blk, DW))

    return pl.pallas_call(
        body,
        grid=(NP // blk,),
        in_specs=[
            pl.BlockSpec((blk, RW1), lambda i: (i, 0)),
            pl.BlockSpec((blk, RW1), lambda i: (i, 0)),
            pl.BlockSpec((8, 64), lambda i: (0, 0)),
            pl.BlockSpec((1, 64), lambda i: (0, 0)),
            pl.BlockSpec((64, 8), lambda i: (0, 0)),
            pl.BlockSpec((1, 8), lambda i: (0, 0)),
            pl.BlockSpec((1, 8), lambda i: (0, 0)),
        ],
        out_specs=[
            pl.BlockSpec((blk, RW2), lambda i: (i, 0)),
            pl.BlockSpec((blk, DW), lambda i: (i, 0)),
        ],
        out_shape=[
            jax.ShapeDtypeStruct((NP, RW2), jnp.float32),
            jax.ShapeDtypeStruct((NP, DW), jnp.float32),
        ],
    )(acc1a, acc1b, rep8, b1, w2, as2, ad2)


def _tc3(acc2a, acc2b, b2):
    blk = 2000

    def body(aa_ref, ab_ref, b_ref, o_ref):
        m = aa_ref[...] + ab_ref[...]
        v = m[:, 0:8] / (m[:, 8:9] + 1e-16) + b_ref[...]
        mx = jnp.max(v, axis=1, keepdims=True)
        lse = mx + jnp.log(jnp.sum(jnp.exp(v - mx), axis=1, keepdims=True))
        o_ref[...] = v - lse

    return pl.pallas_call(
        body,
        grid=(N // blk,),
        in_specs=[
            pl.BlockSpec((blk, RW2), lambda i: (i, 0)),
            pl.BlockSpec((blk, RW2), lambda i: (i, 0)),
            pl.BlockSpec((1, 8), lambda i: (0, 0)),
        ],
        out_specs=pl.BlockSpec((blk, 8), lambda i: (i, 0)),
        out_shape=jax.ShapeDtypeStruct((N, 8), jnp.float32),
    )(acc2a, acc2b, b2)


def kernel(x, edge_index, W1, att_src1, att_dst1, b1,
           W2, att_src2, att_dst2, b2):
    e = edge_index.shape[1]
    tot = e + N
    jpc1, jpc2 = 2, 4
    ch1 = 2 * -(-tot // (2 * NTILES * K * jpc1))
    ch2 = 2 * -(-tot // (2 * NTILES * K * jpc2))
    ep = NTILES * K * max(jpc1 * ch1, jpc2 * ch2)

    er = e // K
    ei3 = edge_index.reshape(2, er, K)

    xp = jnp.pad(x, ((0, NP - N), (0, 0)))
    eye8 = jnp.eye(8, dtype=jnp.float32)
    a1s = (att_src1[:, :, None] * eye8[:, None, :]).reshape(64, 8)
    a1d = (att_dst1[:, :, None] * eye8[:, None, :]).reshape(64, 8)

    t1, d1 = _tc1(xp, W1, a1s, a1d)
    zero1 = jnp.zeros((NP, RW1), jnp.float32)
    acc1a, acc1b = _make_edge_pass(ch1, NP, RW1, jpc1, er, tot)(ei3, t1, d1, zero1)

    rep8 = jnp.repeat(eye8, 8, axis=1)
    t2, d2 = _tc2(acc1a, acc1b, rep8, b1.reshape(1, 64), W2,
                  att_src2.reshape(1, 8), att_dst2.reshape(1, 8))
    zero2 = jnp.zeros((NP, RW2), jnp.float32)
    acc2a, acc2b = _make_edge_pass(ch2, NP, RW2, jpc2, er, tot)(ei3, t2, d2, zero2)

    return _tc3(acc2a, acc2b, b2.reshape(1, 8))


# speedup vs baseline: 204.4811x; 1.0063x over previous
import dataclasses
import functools

import jax
import jax.numpy as jnp
from jax import lax
from jax.experimental import pallas as pl
from jax.experimental.pallas import tpu as pltpu
from jax.experimental.pallas import tpu_sc as plsc

N = 10000
NP = 10240
F_IN = 128
RW1 = 72
RW2 = 16
DW = 16
K = 128
NTILES = 32


def _make_edge_pass(ch, nrow, rw, jpc, er, tot):
    sup = K * jpc
    mesh = plsc.VectorSubcoreMesh(core_axis_name="c", subcore_axis_name="s")
    cp = pltpu.CompilerParams()
    if "needs_layout_passes" in pltpu.CompilerParams.__dataclass_fields__:
        cp = dataclasses.replace(cp, needs_layout_passes=False)
    if "use_tc_tiling_on_sc" in pltpu.CompilerParams.__dataclass_fields__:
        cp = dataclasses.replace(cp, use_tc_tiling_on_sc=False)

    nidx = ch * jpc

    @functools.partial(
        pl.kernel,
        out_type=[jax.ShapeDtypeStruct((nrow, rw), jnp.float32),
                  jax.ShapeDtypeStruct((nrow, rw), jnp.float32)],
        mesh=mesh,
        compiler_params=cp,
        scratch_types=[
            pltpu.VMEM_SHARED((nrow, rw), jnp.float32),
            pltpu.VMEM((nidx, K), jnp.int32),
            pltpu.VMEM((nidx, K), jnp.int32),
            pltpu.VMEM((sup, rw), jnp.float32),
            pltpu.VMEM((sup, rw), jnp.float32),
            pltpu.VMEM((sup, DW), jnp.float32),
            pltpu.VMEM((sup, DW), jnp.float32),
            pltpu.SemaphoreType.DMA,
            pltpu.SemaphoreType.DMA,
            pltpu.SemaphoreType.DMA,
        ],
    )
    def edge_pass(ei_hbm, tbl_hbm, dtbl_hbm, zero_hbm,
                  out_a_hbm, out_b_hbm,
                  acc, sidx, didx, rows_a, rows_b, drows_a, drows_b,
                  gsem, ssem, isem):
        cid = lax.axis_index("c")
        sid = lax.axis_index("s")
        wid = cid * 16 + sid
        iota = lax.iota(jnp.int32, 16)

        zrows = nrow // 16
        pltpu.sync_copy(zero_hbm.at[pl.ds(sid * zrows, zrows)],
                        acc.at[pl.ds(sid * zrows, zrows)])

        @pl.loop(0, nidx)
        def _fill(i):
            r = wid * nidx + i

            @pl.when(r < er)
            def _():
                pltpu.async_copy(ei_hbm.at[0, r], sidx.at[i], isem)
                pltpu.async_copy(ei_hbm.at[1, r], didx.at[i], isem)

            @pl.when(r >= er)
            def _():
                for j in range(8):
                    p = r * K + 16 * j + iota
                    v = jnp.where(p < tot, p - er * K,
                                  N + lax.rem(p, nrow - N))
                    sidx[i, pl.ds(16 * j, 16)] = v
                    didx[i, pl.ds(16 * j, 16)] = v

        @pl.loop(0, nidx)
        def _dr(i):
            r = wid * nidx + i

            @pl.when(r < er)
            def _():
                pltpu.make_async_copy(ei_hbm.at[0, r], sidx.at[i], isem).wait()
                pltpu.make_async_copy(ei_hbm.at[1, r], didx.at[i], isem).wait()

        plsc.subcore_barrier()
        if rw == RW1:
            pats = [8 + iota // 8 + 2 * cc for cc in range(4)]
            pat_tail = (iota + 8) & 15
            m_lt8 = iota < 8
        else:
            splat8 = jnp.zeros((16,), jnp.int32) + 8
            m_lt8 = iota < 8
            m_eq8 = iota == 8
            zero_v = jnp.zeros((16,), jnp.float32)

        def g_fire(g, rows, drows):
            for j in range(jpc):
                r = g * jpc + j
                pltpu.async_copy(tbl_hbm.at[sidx.at[r]],
                                 rows.at[pl.ds(j * K, K)], gsem)
                pltpu.async_copy(dtbl_hbm.at[didx.at[r]],
                                 drows.at[pl.ds(j * K, K)], gsem)

        def g_drain(rows, drows):
            for j in range(jpc):
                pltpu.make_async_copy(tbl_hbm.at[pl.ds(0, K)],
                                      rows.at[pl.ds(j * K, K)], gsem).wait()
                pltpu.make_async_copy(dtbl_hbm.at[pl.ds(0, K)],
                                      drows.at[pl.ds(j * K, K)], gsem).wait()

        def s_start(g, rows):
            for j in range(jpc):
                r = g * jpc + j
                pltpu.async_copy(rows.at[pl.ds(j * K, K)],
                                 acc.at[didx.at[r]], ssem, add=True)

        def s_drain(rows):
            for j in range(jpc):
                pltpu.make_async_copy(rows.at[pl.ds(j * K, K)],
                                      acc.at[pl.ds(0, K)], ssem).wait()

        def compute(rows, drows):
            @plsc.parallel_loop(0, sup, unroll=4)
            def _edge(e):
                dv = drows[e, pl.ds(0, 16)]
                if rw == RW1:
                    av = rows[e, pl.ds(56, 16)]
                    s = av + dv
                    w = jnp.exp(jnp.maximum(s, 0.2 * s))
                    m3 = None
                    for cc in range(4):
                        xwc = rows[e, pl.ds(cc * 16, 16)]
                        wb = w.at[pats[cc]].get(mode="promise_in_bounds")
                        m3 = xwc * wb
                        rows[e, pl.ds(cc * 16, 16)] = m3
                    m3s = m3.at[pat_tail].get(mode="promise_in_bounds")
                    rows[e, pl.ds(56, 16)] = jnp.where(m_lt8, m3s, w)
                else:
                    row = rows[e, pl.ds(0, 16)]
                    sv = row.at[splat8].get(mode="promise_in_bounds") + dv
                    w = jnp.exp(jnp.maximum(sv, 0.2 * sv))
                    out = jnp.where(m_lt8, w * row, jnp.where(m_eq8, w, zero_v))
                    rows[e, pl.ds(0, 16)] = out

        g_fire(0, rows_a, drows_a)

        @pl.loop(0, ch // 2)
        def _pair(t):
            g0 = 2 * t
            g1 = g0 + 1

            @pl.when(t > 0)
            def _():
                s_drain(rows_b)

            g_fire(g1, rows_b, drows_b)
            g_drain(rows_a, drows_a)
            compute(rows_a, drows_a)
            s_start(g0, rows_a)
            g_drain(rows_b, drows_b)
            compute(rows_b, drows_b)
            s_drain(rows_a)

            @pl.when(g1 + 1 < ch)
            def _():
                g_fire(g1 + 1, rows_a, drows_a)

            s_start(g1, rows_b)

        s_drain(rows_b)
        plsc.subcore_barrier()

        @pl.when(cid == 0)
        def _writeout0():
            pltpu.sync_copy(acc.at[pl.ds(sid * zrows, zrows)],
                            out_a_hbm.at[pl.ds(sid * zrows, zrows)])

        @pl.when(cid == 1)
        def _writeout1():
            pltpu.sync_copy(acc.at[pl.ds(sid * zrows, zrows)],
                            out_b_hbm.at[pl.ds(sid * zrows, zrows)])

    return edge_pass


def _tc1(xp, w1, a1s, a1d):
    blk = 2048

    def body(x_ref, w_ref, s_ref, d_ref, t_ref, dd_ref):
        xw = jnp.dot(x_ref[...], w_ref[...], preferred_element_type=jnp.float32)
        asrc = jnp.dot(xw, s_ref[...], preferred_element_type=jnp.float32)
        adst = jnp.dot(xw, d_ref[...], preferred_element_type=jnp.float32)
        z8 = jnp.zeros((blk, 8), jnp.float32)
        t_ref[...] = jnp.concatenate([xw, asrc], axis=1)
        dd_ref[...] = jnp.concatenate([z8, adst], axis=1)

    return pl.pallas_call(
        body,
        grid=(NP // blk,),
        in_specs=[
            pl.BlockSpec((blk, F_IN), lambda i: (i, 0)),
            pl.BlockSpec((F_IN, 64), lambda i: (0, 0)),
            pl.BlockSpec((64, 8), lambda i: (0, 0)),
            pl.BlockSpec((64, 8), lambda i: (0, 0)),
        ],
        out_specs=[
            pl.BlockSpec((blk, RW1), lambda i: (i, 0)),
            pl.BlockSpec((blk, DW), lambda i: (i, 0)),
        ],
        out_shape=[
            jax.ShapeDtypeStruct((NP, RW1), jnp.float32),
            jax.ShapeDtypeStruct((NP, DW), jnp.float32),
        ],
    )(xp, w1, a1s, a1d)


def _tc2(acc1a, acc1b, rep8, b1, w2, as2, ad2):
    blk = 1024

    def body(aa_ref, ab_ref, rep_ref, b_ref, w_ref, s_ref, d_ref, t_ref,
             dd_ref):
        m = aa_ref[...] + ab_ref[...]
        msg = m[:, 0:64]
        den = m[:, 64:72]
        dex = jnp.dot(den, rep_ref[...], preferred_element_type=jnp.float32)
        h = msg / (dex + 1e-16) + b_ref[...]
        h = jnp.where(h > 0, h, jnp.exp(h) - 1.0)
        xw2 = jnp.dot(h, w_ref[...], preferred_element_type=jnp.float32)
        asrc2 = jnp.sum(xw2 * s_ref[...], axis=1, keepdims=True)
        adst2 = jnp.sum(xw2 * d_ref[...], axis=1, keepdims=True)
        z7 = jnp.zeros((blk, 7), jnp.float32)
        t_ref[...] = jnp.concatenate([xw2, asrc2, z7], axis=1)
        dd_ref[...] = jnp.broadcast_to(adst2, (blk, DW))

    return pl.pallas_call(
        body,
        grid=(NP // blk,),
        in_specs=[
            pl.BlockSpec((blk, RW1), lambda i: (i, 0)),
            pl.BlockSpec((blk, RW1), lambda i: (i, 0)),
            pl.BlockSpec((8, 64), lambda i: (0, 0)),
            pl.BlockSpec((1, 64), lambda i: (0, 0)),
            pl.BlockSpec((64, 8), lambda i: (0, 0)),
            pl.BlockSpec((1, 8), lambda i: (0, 0)),
            pl.BlockSpec((1, 8), lambda i: (0, 0)),
        ],
        out_specs=[
            pl.BlockSpec((blk, RW2), lambda i: (i, 0)),
            pl.BlockSpec((blk, DW), lambda i: (i, 0)),
        ],
        out_shape=[
            jax.ShapeDtypeStruct((NP, RW2), jnp.float32),
            jax.ShapeDtypeStruct((NP, DW), jnp.float32),
        ],
    )(acc1a, acc1b, rep8, b1, w2, as2, ad2)


def _tc3(acc2a, acc2b, b2):
    blk = 2000

    def body(aa_ref, ab_ref, b_ref, o_ref):
        m = aa_ref[...] + ab_ref[...]
        v = m[:, 0:8] / (m[:, 8:9] + 1e-16) + b_ref[...]
        mx = jnp.max(v, axis=1, keepdims=True)
        lse = mx + jnp.log(jnp.sum(jnp.exp(v - mx), axis=1, keepdims=True))
        o_ref[...] = v - lse

    return pl.pallas_call(
        body,
        grid=(N // blk,),
        in_specs=[
            pl.BlockSpec((blk, RW2), lambda i: (i, 0)),
            pl.BlockSpec((blk, RW2), lambda i: (i, 0)),
            pl.BlockSpec((1, 8), lambda i: (0, 0)),
        ],
        out_specs=pl.BlockSpec((blk, 8), lambda i: (i, 0)),
        out_shape=jax.ShapeDtypeStruct((N, 8), jnp.float32),
    )(acc2a, acc2b, b2)


def kernel(x, edge_index, W1, att_src1, att_dst1, b1,
           W2, att_src2, att_dst2, b2):
    e = edge_index.shape[1]
    tot = e + N
    jpc1, jpc2 = 2, 4
    ch1 = 2 * -(-tot // (2 * NTILES * K * jpc1))
    ch2 = 2 * -(-tot // (2 * NTILES * K * jpc2))
    ep = NTILES * K * max(jpc1 * ch1, jpc2 * ch2)

    er = e // K
    ei3 = edge_index.reshape(2, er, K)

    xp = jnp.pad(x, ((0, NP - N), (0, 0)))
    eye8 = jnp.eye(8, dtype=jnp.float32)
    a1s = (att_src1[:, :, None] * eye8[:, None, :]).reshape(64, 8)
    a1d = (att_dst1[:, :, None] * eye8[:, None, :]).reshape(64, 8)

    t1, d1 = _tc1(xp, W1, a1s, a1d)
    zero1 = jnp.zeros((NP, RW1), jnp.float32)
    acc1a, acc1b = _make_edge_pass(ch1, NP, RW1, jpc1, er, tot)(ei3, t1, d1, zero1)

    rep8 = jnp.repeat(eye8, 8, axis=1)
    t2, d2 = _tc2(acc1a, acc1b, rep8, b1.reshape(1, 64), W2,
                  att_src2.reshape(1, 8), att_dst2.reshape(1, 8))
    zero2 = jnp.zeros((NP, RW2), jnp.float32)
    acc2a, acc2b = _make_edge_pass(ch2, NP, RW2, jpc2, er, tot)(ei3, t2, d2, zero2)

    return _tc3(acc2a, acc2b, b2.reshape(1, 8))
